# baseline (speedup 1.0000x reference)
import functools

import jax
import jax.numpy as jnp
from jax import lax
from jax.experimental import pallas as pl
from jax.experimental.pallas import tpu as pltpu
from jax.experimental.pallas import tpu_sc as plsc

NC = 2
NS = 16
NW = NC * NS
CHUNK = 128
SUP = 16


def _sc_agg(x, src2, dst2, batch, attr, zs, zc, ones):
    n_src, d = x.shape
    n_chunks = src2.shape[0]
    chunks_per_tile = n_chunks // NW
    n_sup = chunks_per_tile // SUP
    n_dst = zs.shape[0]
    stripe = 256
    s_tail = n_dst - (NS - 1) * stripe
    a_tail = n_dst - (NW - 1) * 128

    mesh = plsc.VectorSubcoreMesh(core_axis_name="c", subcore_axis_name="s")

    @functools.partial(
        pl.kernel,
        out_type=(
            jax.ShapeDtypeStruct((NC, n_dst, d), jnp.float32),
            jax.ShapeDtypeStruct((NC, n_dst, d), jnp.float32),
            jax.ShapeDtypeStruct((n_dst, d), jnp.float32),
        ),
        mesh=mesh,
        scratch_types=(
            pltpu.VMEM((SUP, CHUNK), jnp.int32),
            pltpu.VMEM((SUP, CHUNK), jnp.int32),
            pltpu.VMEM((CHUNK, d), jnp.float32),
            pltpu.VMEM((CHUNK, d), jnp.float32),
            pltpu.VMEM((CHUNK, d), jnp.float32),
            pltpu.SemaphoreType.DMA,
            pltpu.SemaphoreType.DMA,
            pltpu.SemaphoreType.DMA,
            pltpu.SemaphoreType.DMA,
            pltpu.VMEM_SHARED((n_dst, d), jnp.float32),
            pltpu.VMEM_SHARED((n_dst, d), jnp.float32),
        ),
    )
    def body(x_h, src_h, dst_h, batch_h, attr_h, zs_h, zc_h, ones_h,
             summed_o, cnt_o, attr_o,
             src_v, dst_v, rows0, rows1, ones_v,
             g0, g1, s0, s1, summed_sh, cnt_sh):
        c = lax.axis_index("c")
        s = lax.axis_index("s")
        wid = s * NC + c
        rows = (rows0, rows1)
        gsem = (g0, g1)
        ssem = (s0, s1)

        r0 = pl.multiple_of(s * stripe, stripe)

        @pl.when(s < NS - 1)
        def _zero_full():
            pltpu.sync_copy(zs_h.at[pl.ds(r0, stripe)],
                            summed_sh.at[pl.ds(r0, stripe)])
            pltpu.sync_copy(zc_h.at[pl.ds(r0, stripe)],
                            cnt_sh.at[pl.ds(r0, stripe)])

        @pl.when(s == NS - 1)
        def _zero_tail():
            t0 = (NS - 1) * stripe
            pltpu.sync_copy(zs_h.at[pl.ds(t0, s_tail)],
                            summed_sh.at[pl.ds(t0, s_tail)])
            pltpu.sync_copy(zc_h.at[pl.ds(t0, s_tail)],
                            cnt_sh.at[pl.ds(t0, s_tail)])

        pltpu.sync_copy(ones_h, ones_v)
        plsc.subcore_barrier()

        c0 = wid * chunks_per_tile

        def chunk_body(i, carry):
            pltpu.sync_copy(src_h.at[i], src_v.at[0])
            pltpu.sync_copy(dst_h.at[i], dst_v.at[0])
            pltpu.async_copy(x_h.at[src_v.at[0]], rows0, g0).wait()
            pltpu.sync_copy(rows0, summed_sh.at[dst_v.at[0]], add=True)
            pltpu.sync_copy(ones_v, cnt_sh.at[dst_v.at[0]], add=True)
            return carry

        lax.fori_loop(c0, c0 + chunks_per_tile, chunk_body, 0)
        plsc.subcore_barrier()

        @pl.when(s < NS - 1)
        def _pub_full():
            pltpu.sync_copy(summed_sh.at[pl.ds(r0, stripe)],
                            summed_o.at[c, pl.ds(r0, stripe)])
            pltpu.sync_copy(cnt_sh.at[pl.ds(r0, stripe)],
                            cnt_o.at[c, pl.ds(r0, stripe)])

        @pl.when(s == NS - 1)
        def _pub_tail():
            t0 = (NS - 1) * stripe
            pltpu.sync_copy(summed_sh.at[pl.ds(t0, s_tail)],
                            summed_o.at[c, pl.ds(t0, s_tail)])
            pltpu.sync_copy(cnt_sh.at[pl.ds(t0, s_tail)],
                            cnt_o.at[c, pl.ds(t0, s_tail)])

        @pl.when(wid < NW - 1)
        def _full():
            b = pl.multiple_of(wid * 128, 128)
            pltpu.sync_copy(batch_h.at[pl.ds(b, 128)], src_v.at[0])
            pltpu.async_copy(attr_h.at[src_v.at[0]], rows0, g0).wait()
            pltpu.sync_copy(rows0, attr_o.at[pl.ds(b, 128)])

        @pl.when(wid == NW - 1)
        def _tail():
            b = (NW - 1) * 128
            pltpu.sync_copy(batch_h.at[pl.ds(b, a_tail)],
                            src_v.at[0, pl.ds(0, a_tail)])
            pltpu.async_copy(attr_h.at[src_v.at[0, pl.ds(0, a_tail)]],
                             rows0.at[pl.ds(0, a_tail)], g0).wait()
            pltpu.sync_copy(rows0.at[pl.ds(0, a_tail)],
                            attr_o.at[pl.ds(b, a_tail)])

    return body(x, src2, dst2, batch, attr, zs, zc, ones)


def _tc_combine(summed2, cnt2, x_t, attr_g, W_l, W_r, W_lin, b_l, b_lin,
                n_pad):
    n_dst, d = x_t.shape
    blk = 1000
    grid = n_dst // blk
    dn = (((1,), (1,)), ((), ()))
    fpad = float(n_pad)

    def body(s2, c2, xt, ag, wl, wr, wlin, bl, blin, o):
        ssum = s2[0] + s2[1]
        cnt = c2[0] + c2[1]
        row = lax.broadcasted_iota(jnp.int32, (blk, 1), 0)
        corr = jnp.where((row == 0) & (pl.program_id(0) == 0), fpad, 0.0)
        ssum = ssum - corr * xt[0:1, :]
        mean = ssum / jnp.maximum(cnt[:, 0:1] - corr, 1.0)
        acc = lax.dot_general(mean, wl[...], dn,
                              preferred_element_type=jnp.float32)
        acc = acc + lax.dot_general(xt[...], wr[...], dn,
                                    preferred_element_type=jnp.float32)
        acc = acc + 0.25 * lax.dot_general(ag[...], wlin[...], dn,
                                           preferred_element_type=jnp.float32)
        acc = acc + (bl[...] + 0.25 * blin[...])
        o[...] = jnp.maximum(acc, 0.0)

    return pl.pallas_call(
        body,
        grid=(grid,),
        in_specs=[
            pl.BlockSpec((NC, blk, d), lambda i: (0, i, 0)),
            pl.BlockSpec((NC, blk, d), lambda i: (0, i, 0)),
            pl.BlockSpec((blk, d), lambda i: (i, 0)),
            pl.BlockSpec((blk, d), lambda i: (i, 0)),
            pl.BlockSpec((d, d), lambda i: (0, 0)),
            pl.BlockSpec((d, d), lambda i: (0, 0)),
            pl.BlockSpec((d, d), lambda i: (0, 0)),
            pl.BlockSpec((1, d), lambda i: (0, 0)),
            pl.BlockSpec((1, d), lambda i: (0, 0)),
        ],
        out_specs=pl.BlockSpec((blk, d), lambda i: (i, 0)),
        out_shape=jax.ShapeDtypeStruct((n_dst, d), jnp.float32),
    )(summed2, cnt2, x_t, attr_g, W_l, W_r, W_lin, b_l, b_lin)


def kernel(x, edge_index, batch, attr, W_l, b_l, W_r, W_lin, b_lin,
           size_src, size_dst):
    src = edge_index[0]
    dst = edge_index[1]
    n_dst = batch.shape[0]
    e = src.shape[0]
    step = NW * CHUNK * 8
    e_pad = -(-e // step) * step
    n_pad = e_pad - e
    pad = jnp.zeros((n_pad,), jnp.int32)
    src2 = jnp.concatenate([src, pad]).reshape(e_pad // CHUNK, CHUNK)
    dst2 = jnp.concatenate([dst, pad]).reshape(e_pad // CHUNK, CHUNK)
    zs = jnp.zeros((n_dst, x.shape[1]), jnp.float32)
    zc = jnp.zeros((n_dst, x.shape[1]), jnp.float32)
    ones = jnp.ones((CHUNK, x.shape[1]), jnp.float32)
    summed2, cnt2, attr_g = _sc_agg(x, src2, dst2, batch, attr, zs, zc, ones)
    return _tc_combine(summed2, cnt2, x[:n_dst], attr_g, W_l, W_r, W_lin,
                       b_l.reshape(1, -1), b_lin.reshape(1, -1), n_pad)


# speedup vs baseline: 2.3943x; 2.3943x over previous
import functools

import jax
import jax.numpy as jnp
from jax import lax
from jax.experimental import pallas as pl
from jax.experimental.pallas import tpu as pltpu
from jax.experimental.pallas import tpu_sc as plsc

NC = 2
NS = 16
NW = NC * NS
CHUNK = 128


def _sc_agg(x, src, dst, batch, attr, zs):
    n_src, d = x.shape
    e = src.shape[0]
    n_dst = zs.shape[0]
    n_chunks = e // CHUNK
    stripe = 256
    s_tail = n_dst - (NS - 1) * stripe
    a_tail = n_dst - (NW - 1) * 128

    mesh = plsc.VectorSubcoreMesh(core_axis_name="c", subcore_axis_name="s")

    @functools.partial(
        pl.kernel,
        out_type=(
            jax.ShapeDtypeStruct((NC, n_dst, d), jnp.float32),
            jax.ShapeDtypeStruct((NC, n_dst, 8), jnp.float32),
            jax.ShapeDtypeStruct((n_dst, d), jnp.float32),
        ),
        mesh=mesh,
        scratch_types=(
            pltpu.VMEM((CHUNK,), jnp.int32),
            pltpu.VMEM((CHUNK,), jnp.int32),
            pltpu.VMEM((CHUNK, d), jnp.float32),
            pltpu.VMEM((n_dst,), jnp.float32),
            pltpu.VMEM((NS * stripe,), jnp.float32),
            pltpu.VMEM((stripe, 8), jnp.float32),
            pltpu.SemaphoreType.DMA,
            pltpu.VMEM_SHARED((n_dst, d), jnp.float32),
        ) + tuple(pltpu.VMEM_SHARED((n_dst,), jnp.float32)
                  for _ in range(NS)),
        compiler_params=pltpu.CompilerParams(needs_layout_passes=False),
    )
    def body(x_h, src_h, dst_h, batch_h, attr_h, zs_h,
             summed_o, cnt_o, attr_o,
             src_v, dst_v, rows_v, hist_v, slab_v, col_v, sem,
             summed_sh, *slots_sh):
        c = lax.axis_index("c")
        s = lax.axis_index("s")
        wid = s * NC + c

        r0 = pl.multiple_of(s * stripe, stripe)

        @pl.when(s < NS - 1)
        def _zero_full():
            pltpu.sync_copy(zs_h.at[pl.ds(r0, stripe)],
                            summed_sh.at[pl.ds(r0, stripe)])

        @pl.when(s == NS - 1)
        def _zero_tail():
            t0 = (NS - 1) * stripe
            pltpu.sync_copy(zs_h.at[pl.ds(t0, s_tail)],
                            summed_sh.at[pl.ds(t0, s_tail)])

        z16 = jnp.zeros((16,), jnp.float32)

        def zero_body(i, carry):
            hist_v[pl.ds(pl.multiple_of(i * 16, 16), 16)] = z16
            return carry

        lax.fori_loop(0, n_dst // 16, zero_body, 0)
        plsc.subcore_barrier()

        c0 = (n_chunks * wid) // NW
        c1 = (n_chunks * (wid + 1)) // NW
        one16 = jnp.ones((16,), jnp.float32)

        def chunk_body(i, carry):
            base = pl.multiple_of(i * CHUNK, CHUNK)
            pltpu.sync_copy(src_h.at[pl.ds(base, CHUNK)], src_v)
            pltpu.sync_copy(dst_h.at[pl.ds(base, CHUNK)], dst_v)
            pltpu.async_copy(x_h.at[src_v], rows_v, sem).wait()
            pltpu.sync_copy(rows_v, summed_sh.at[dst_v], add=True)
            for j in range(CHUNK // 16):
                dv = dst_v[pl.ds(j * 16, 16)]
                plsc.addupdate_scatter(hist_v, [dv], one16)
            return carry

        lax.fori_loop(c0, c1, chunk_body, 0)
        for k in range(NS):
            @pl.when(s == k)
            def _pub_hist():
                pltpu.sync_copy(hist_v, slots_sh[k])
        plsc.subcore_barrier()

        def pub(r_lo, sz):
            pltpu.sync_copy(summed_sh.at[pl.ds(r_lo, sz)],
                            summed_o.at[c, pl.ds(r_lo, sz)])
            for k in range(NS):
                pltpu.sync_copy(slots_sh[k].at[pl.ds(r_lo, sz)],
                                slab_v.at[pl.ds(k * stripe, sz)])
            for v in range(sz // 16):
                acc = slab_v[pl.ds(v * 16, 16)]
                for k in range(1, NS):
                    acc = acc + slab_v[pl.ds(k * stripe + v * 16, 16)]
                ridx = v * 16 + lax.iota(jnp.int32, 16)
                plsc.store_scatter(col_v, [ridx, jnp.zeros((16,), jnp.int32)],
                                   acc)
            pltpu.sync_copy(col_v.at[pl.ds(0, sz)],
                            cnt_o.at[c, pl.ds(r_lo, sz)])

        @pl.when(s < NS - 1)
        def _pub_full():
            pub(r0, stripe)

        @pl.when(s == NS - 1)
        def _pub_tail():
            pub((NS - 1) * stripe, s_tail)

        @pl.when(wid < NW - 1)
        def _full():
            b = pl.multiple_of(wid * 128, 128)
            pltpu.sync_copy(batch_h.at[pl.ds(b, 128)], src_v)
            pltpu.async_copy(attr_h.at[src_v], rows_v, sem).wait()
            pltpu.sync_copy(rows_v, attr_o.at[pl.ds(b, 128)])

        @pl.when(wid == NW - 1)
        def _tail():
            b = (NW - 1) * 128
            pltpu.sync_copy(batch_h.at[pl.ds(b, a_tail)],
                            src_v.at[pl.ds(0, a_tail)])
            pltpu.async_copy(attr_h.at[src_v.at[pl.ds(0, a_tail)]],
                             rows_v.at[pl.ds(0, a_tail)], sem).wait()
            pltpu.sync_copy(rows_v.at[pl.ds(0, a_tail)],
                            attr_o.at[pl.ds(b, a_tail)])

    return body(x, src, dst, batch, attr, zs)


def _tc_combine(summed2, cnt2, x_t, attr_g, W_l, W_r, W_lin, b_l, b_lin):
    n_dst, d = x_t.shape
    blk = 1000
    grid = n_dst // blk
    dn = (((1,), (1,)), ((), ()))

    def body(s2, c2, xt, ag, wl, wr, wlin, bl, blin, o):
        ssum = s2[0] + s2[1]
        cnt = c2[0] + c2[1]
        mean = ssum / jnp.maximum(cnt[:, 0:1], 1.0)
        acc = lax.dot_general(mean, wl[...], dn,
                              preferred_element_type=jnp.float32)
        acc = acc + lax.dot_general(xt[...], wr[...], dn,
                                    preferred_element_type=jnp.float32)
        acc = acc + 0.25 * lax.dot_general(ag[...], wlin[...], dn,
                                           preferred_element_type=jnp.float32)
        acc = acc + (bl[...] + 0.25 * blin[...])
        o[...] = jnp.maximum(acc, 0.0)

    return pl.pallas_call(
        body,
        grid=(grid,),
        in_specs=[
            pl.BlockSpec((NC, blk, d), lambda i: (0, i, 0)),
            pl.BlockSpec((NC, blk, 8), lambda i: (0, i, 0)),
            pl.BlockSpec((blk, d), lambda i: (i, 0)),
            pl.BlockSpec((blk, d), lambda i: (i, 0)),
            pl.BlockSpec((d, d), lambda i: (0, 0)),
            pl.BlockSpec((d, d), lambda i: (0, 0)),
            pl.BlockSpec((d, d), lambda i: (0, 0)),
            pl.BlockSpec((1, d), lambda i: (0, 0)),
            pl.BlockSpec((1, d), lambda i: (0, 0)),
        ],
        out_specs=pl.BlockSpec((blk, d), lambda i: (i, 0)),
        out_shape=jax.ShapeDtypeStruct((n_dst, d), jnp.float32),
    )(summed2, cnt2, x_t, attr_g, W_l, W_r, W_lin, b_l, b_lin)


def kernel(x, edge_index, batch, attr, W_l, b_l, W_r, W_lin, b_lin,
           size_src, size_dst):
    src = edge_index[0]
    dst = edge_index[1]
    n_dst = batch.shape[0]
    zs = jnp.zeros((n_dst, x.shape[1]), jnp.float32)
    summed2, cnt2, attr_g = _sc_agg(x, src, dst, batch, attr, zs)
    return _tc_combine(summed2, cnt2, x[:n_dst], attr_g, W_l, W_r, W_lin,
                       b_l.reshape(1, -1), b_lin.reshape(1, -1))


# speedup vs baseline: 3.5839x; 1.4968x over previous
import functools

import jax
import jax.numpy as jnp
from jax import lax
from jax.experimental import pallas as pl
from jax.experimental.pallas import tpu as pltpu
from jax.experimental.pallas import tpu_sc as plsc

NC = 2
NS = 16
NW = NC * NS
CHUNK = 128


def _sc_agg(x, src, dst, batch, attr, zs):
    n_src, d = x.shape
    e = src.shape[0]
    n_dst = zs.shape[0]
    n_chunks = e // CHUNK
    stripe = 256
    s_tail = n_dst - (NS - 1) * stripe
    a_tail = n_dst - (NW - 1) * 128

    mesh = plsc.VectorSubcoreMesh(core_axis_name="c", subcore_axis_name="s")

    @functools.partial(
        pl.kernel,
        out_type=(
            jax.ShapeDtypeStruct((NC, n_dst, d), jnp.float32),
            jax.ShapeDtypeStruct((NC, n_dst, 8), jnp.float32),
            jax.ShapeDtypeStruct((n_dst, d), jnp.float32),
        ),
        mesh=mesh,
        scratch_types=(
            pltpu.VMEM((CHUNK,), jnp.int32),
            pltpu.VMEM((CHUNK,), jnp.int32),
            pltpu.VMEM((CHUNK,), jnp.int32),
            pltpu.VMEM((CHUNK,), jnp.int32),
            pltpu.VMEM((CHUNK, d), jnp.float32),
            pltpu.VMEM((CHUNK, d), jnp.float32),
            pltpu.VMEM((n_dst,), jnp.float32),
            pltpu.VMEM((NS * stripe,), jnp.float32),
            pltpu.VMEM((stripe, 8), jnp.float32),
            pltpu.SemaphoreType.DMA,
            pltpu.SemaphoreType.DMA,
            pltpu.VMEM_SHARED((n_dst, d), jnp.float32),
        ) + tuple(pltpu.VMEM_SHARED((n_dst,), jnp.float32)
                  for _ in range(NS)),
        compiler_params=pltpu.CompilerParams(needs_layout_passes=False),
    )
    def body(x_h, src_h, dst_h, batch_h, attr_h, zs_h,
             summed_o, cnt_o, attr_o,
             src_a, src_b, dst_a, dst_b, rows_a, rows_b,
             hist_v, slab_v, col_v, sem_a, sem_b,
             summed_sh, *slots_sh):
        src_v, dst_v, rows_v, sem = src_a, dst_a, rows_a, sem_a
        srcs = (src_a, src_b)
        dsts = (dst_a, dst_b)
        rows2 = (rows_a, rows_b)
        sems = (sem_a, sem_b)
        c = lax.axis_index("c")
        s = lax.axis_index("s")
        wid = s * NC + c

        r0 = pl.multiple_of(s * stripe, stripe)

        @pl.when(s < NS - 1)
        def _zero_full():
            pltpu.sync_copy(zs_h.at[pl.ds(r0, stripe)],
                            summed_sh.at[pl.ds(r0, stripe)])

        @pl.when(s == NS - 1)
        def _zero_tail():
            t0 = (NS - 1) * stripe
            pltpu.sync_copy(zs_h.at[pl.ds(t0, s_tail)],
                            summed_sh.at[pl.ds(t0, s_tail)])

        z16 = jnp.zeros((16,), jnp.float32)

        def zero_body(i, carry):
            hist_v[pl.ds(pl.multiple_of(i * 16, 16), 16)] = z16
            return carry

        lax.fori_loop(0, n_dst // 16, zero_body, 0)
        plsc.subcore_barrier()

        c0 = (n_chunks * wid) // NW
        c1 = (n_chunks * (wid + 1)) // NW
        one16 = jnp.ones((16,), jnp.float32)

        def load_and_fire(i, b):
            base = pl.multiple_of(i * CHUNK, CHUNK)
            pltpu.sync_copy(src_h.at[pl.ds(base, CHUNK)], srcs[b])
            pltpu.sync_copy(dst_h.at[pl.ds(base, CHUNK)], dsts[b])
            pltpu.async_copy(x_h.at[srcs[b]], rows2[b], sems[b])

        load_and_fire(c0, 0)
        load_and_fire(c0 + 1, 1)

        def chunk_body(i, carry):
            for b in (0, 1):
                @pl.when((i - c0) % 2 == b)
                def _():
                    pltpu.make_async_copy(x_h.at[srcs[b]], rows2[b],
                                          sems[b]).wait()
                    for j in range(CHUNK // 16):
                        dv = dsts[b][pl.ds(j * 16, 16)]
                        plsc.addupdate_scatter(hist_v, [dv], one16)
                    pltpu.sync_copy(rows2[b], summed_sh.at[dsts[b]],
                                    add=True)

                    @pl.when(i + 2 < c1)
                    def _pref():
                        load_and_fire(i + 2, b)
            return carry

        lax.fori_loop(c0, c1, chunk_body, 0)
        for k in range(NS):
            @pl.when(s == k)
            def _pub_hist():
                pltpu.sync_copy(hist_v, slots_sh[k])
        plsc.subcore_barrier()

        def pub(r_lo, sz):
            pltpu.sync_copy(summed_sh.at[pl.ds(r_lo, sz)],
                            summed_o.at[c, pl.ds(r_lo, sz)])
            for k in range(NS):
                pltpu.sync_copy(slots_sh[k].at[pl.ds(r_lo, sz)],
                                slab_v.at[pl.ds(k * stripe, sz)])
            for v in range(sz // 16):
                acc = slab_v[pl.ds(v * 16, 16)]
                for k in range(1, NS):
                    acc = acc + slab_v[pl.ds(k * stripe + v * 16, 16)]
                ridx = v * 16 + lax.iota(jnp.int32, 16)
                plsc.store_scatter(col_v, [ridx, jnp.zeros((16,), jnp.int32)],
                                   acc)
            pltpu.sync_copy(col_v.at[pl.ds(0, sz)],
                            cnt_o.at[c, pl.ds(r_lo, sz)])

        @pl.when(s < NS - 1)
        def _pub_full():
            pub(r0, stripe)

        @pl.when(s == NS - 1)
        def _pub_tail():
            pub((NS - 1) * stripe, s_tail)

        @pl.when(wid < NW - 1)
        def _full():
            b = pl.multiple_of(wid * 128, 128)
            pltpu.sync_copy(batch_h.at[pl.ds(b, 128)], src_v)
            pltpu.async_copy(attr_h.at[src_v], rows_v, sem).wait()
            pltpu.sync_copy(rows_v, attr_o.at[pl.ds(b, 128)])

        @pl.when(wid == NW - 1)
        def _tail():
            b = (NW - 1) * 128
            pltpu.sync_copy(batch_h.at[pl.ds(b, a_tail)],
                            src_v.at[pl.ds(0, a_tail)])
            pltpu.async_copy(attr_h.at[src_v.at[pl.ds(0, a_tail)]],
                             rows_v.at[pl.ds(0, a_tail)], sem).wait()
            pltpu.sync_copy(rows_v.at[pl.ds(0, a_tail)],
                            attr_o.at[pl.ds(b, a_tail)])

    return body(x, src, dst, batch, attr, zs)


def _tc_combine(summed2, cnt2, x_t, attr_g, W_l, W_r, W_lin, b_l, b_lin):
    n_dst, d = x_t.shape
    blk = 1000
    grid = n_dst // blk
    dn = (((1,), (1,)), ((), ()))

    def body(s2, c2, xt, ag, wl, wr, wlin, bl, blin, o):
        ssum = s2[0] + s2[1]
        cnt = c2[0] + c2[1]
        mean = ssum / jnp.maximum(cnt[:, 0:1], 1.0)
        acc = lax.dot_general(mean, wl[...], dn,
                              preferred_element_type=jnp.float32)
        acc = acc + lax.dot_general(xt[...], wr[...], dn,
                                    preferred_element_type=jnp.float32)
        acc = acc + 0.25 * lax.dot_general(ag[...], wlin[...], dn,
                                           preferred_element_type=jnp.float32)
        acc = acc + (bl[...] + 0.25 * blin[...])
        o[...] = jnp.maximum(acc, 0.0)

    return pl.pallas_call(
        body,
        grid=(grid,),
        in_specs=[
            pl.BlockSpec((NC, blk, d), lambda i: (0, i, 0)),
            pl.BlockSpec((NC, blk, 8), lambda i: (0, i, 0)),
            pl.BlockSpec((blk, d), lambda i: (i, 0)),
            pl.BlockSpec((blk, d), lambda i: (i, 0)),
            pl.BlockSpec((d, d), lambda i: (0, 0)),
            pl.BlockSpec((d, d), lambda i: (0, 0)),
            pl.BlockSpec((d, d), lambda i: (0, 0)),
            pl.BlockSpec((1, d), lambda i: (0, 0)),
            pl.BlockSpec((1, d), lambda i: (0, 0)),
        ],
        out_specs=pl.BlockSpec((blk, d), lambda i: (i, 0)),
        out_shape=jax.ShapeDtypeStruct((n_dst, d), jnp.float32),
    )(summed2, cnt2, x_t, attr_g, W_l, W_r, W_lin, b_l, b_lin)


def kernel(x, edge_index, batch, attr, W_l, b_l, W_r, W_lin, b_lin,
           size_src, size_dst):
    src = edge_index[0]
    dst = edge_index[1]
    n_dst = batch.shape[0]
    zs = jnp.zeros((n_dst, x.shape[1]), jnp.float32)
    summed2, cnt2, attr_g = _sc_agg(x, src, dst, batch, attr, zs)
    return _tc_combine(summed2, cnt2, x[:n_dst], attr_g, W_l, W_r, W_lin,
                       b_l.reshape(1, -1), b_lin.reshape(1, -1))


# speedup vs baseline: 4.2696x; 1.1913x over previous
import functools

import jax
import jax.numpy as jnp
from jax import lax
from jax.experimental import pallas as pl
from jax.experimental.pallas import tpu as pltpu
from jax.experimental.pallas import tpu_sc as plsc

NC = 2
NS = 16
NW = NC * NS
CHUNK = 128


def _sc_agg(x, src, dst, batch, attr, zs):
    n_src, d = x.shape
    e = src.shape[0]
    n_dst = zs.shape[0]
    n_chunks = e // CHUNK
    stripe = 256
    s_tail = n_dst - (NS - 1) * stripe
    a_tail = n_dst - (NW - 1) * 128

    mesh = plsc.VectorSubcoreMesh(core_axis_name="c", subcore_axis_name="s")

    @functools.partial(
        pl.kernel,
        out_type=(
            jax.ShapeDtypeStruct((NC, n_dst, d), jnp.float32),
            jax.ShapeDtypeStruct((NC, n_dst, 8), jnp.float32),
            jax.ShapeDtypeStruct((n_dst, d), jnp.float32),
        ),
        mesh=mesh,
        scratch_types=(
            pltpu.VMEM((CHUNK,), jnp.int32),
            pltpu.VMEM((CHUNK,), jnp.int32),
            pltpu.VMEM((CHUNK,), jnp.int32),
            pltpu.VMEM((CHUNK,), jnp.int32),
            pltpu.VMEM((CHUNK,), jnp.int32),
            pltpu.VMEM((CHUNK,), jnp.int32),
            pltpu.VMEM((CHUNK, d), jnp.float32),
            pltpu.VMEM((CHUNK, d), jnp.float32),
            pltpu.VMEM((CHUNK, d), jnp.float32),
            pltpu.VMEM((n_dst,), jnp.float32),
            pltpu.VMEM((NS * stripe,), jnp.float32),
            pltpu.VMEM((stripe, 8), jnp.float32),
            pltpu.SemaphoreType.DMA,
            pltpu.SemaphoreType.DMA,
            pltpu.SemaphoreType.DMA,
            pltpu.SemaphoreType.DMA,
            pltpu.SemaphoreType.DMA,
            pltpu.SemaphoreType.DMA,
            pltpu.VMEM_SHARED((n_dst, d), jnp.float32),
        ) + tuple(pltpu.VMEM_SHARED((n_dst,), jnp.float32)
                  for _ in range(NS)),
        compiler_params=pltpu.CompilerParams(needs_layout_passes=False),
    )
    def body(x_h, src_h, dst_h, batch_h, attr_h, zs_h,
             summed_o, cnt_o, attr_o,
             src_a, src_b, src_c, dst_a, dst_b, dst_c,
             rows_a, rows_b, rows_c,
             hist_v, slab_v, col_v, g_a, g_b, g_c, s_a, s_b, s_c,
             summed_sh, *slots_sh):
        src_v, dst_v, rows_v, sem = src_a, dst_a, rows_a, g_a
        srcs = (src_a, src_b, src_c)
        dsts = (dst_a, dst_b, dst_c)
        rows2 = (rows_a, rows_b, rows_c)
        gsems = (g_a, g_b, g_c)
        ssems = (s_a, s_b, s_c)
        c = lax.axis_index("c")
        s = lax.axis_index("s")
        wid = s * NC + c

        r0 = pl.multiple_of(s * stripe, stripe)

        @pl.when(s < NS - 1)
        def _zero_full():
            pltpu.sync_copy(zs_h.at[pl.ds(r0, stripe)],
                            summed_sh.at[pl.ds(r0, stripe)])

        @pl.when(s == NS - 1)
        def _zero_tail():
            t0 = (NS - 1) * stripe
            pltpu.sync_copy(zs_h.at[pl.ds(t0, s_tail)],
                            summed_sh.at[pl.ds(t0, s_tail)])

        z16 = jnp.zeros((16,), jnp.float32)

        def zero_body(i, carry):
            hist_v[pl.ds(pl.multiple_of(i * 16, 16), 16)] = z16
            return carry

        lax.fori_loop(0, n_dst // 16, zero_body, 0)
        plsc.subcore_barrier()

        c0 = (n_chunks * wid) // NW
        c1 = (n_chunks * (wid + 1)) // NW
        one16 = jnp.ones((16,), jnp.float32)

        def load_and_fire(i, b):
            base = pl.multiple_of(i * CHUNK, CHUNK)
            pltpu.sync_copy(src_h.at[pl.ds(base, CHUNK)], srcs[b])
            pltpu.sync_copy(dst_h.at[pl.ds(base, CHUNK)], dsts[b])
            pltpu.async_copy(x_h.at[srcs[b]], rows2[b], gsems[b])

        def wait_scatter(b):
            pltpu.make_async_copy(rows2[b], summed_sh.at[dsts[b]],
                                  ssems[b]).wait()

        load_and_fire(c0, 0)
        load_and_fire(c0 + 1, 1)

        def chunk_body(i, carry):
            for b in (0, 1, 2):
                @pl.when((i - c0) % 3 == b)
                def _():
                    pltpu.make_async_copy(x_h.at[srcs[b]], rows2[b],
                                          gsems[b]).wait()
                    for j in range(CHUNK // 16):
                        dv = dsts[b][pl.ds(j * 16, 16)]
                        plsc.addupdate_scatter(hist_v, [dv], one16)
                    pltpu.async_copy(rows2[b], summed_sh.at[dsts[b]],
                                     ssems[b], add=True)
                    bn = (b + 2) % 3

                    @pl.when(i > c0)
                    def _drain():
                        wait_scatter(bn)

                    @pl.when(i + 2 < c1)
                    def _pref():
                        load_and_fire(i + 2, bn)
            return carry

        lax.fori_loop(c0, c1, chunk_body, 0)
        for b in (0, 1, 2):
            @pl.when((c1 - 1 - c0) % 3 == b)
            def _final_drain():
                wait_scatter(b)
        for k in range(NS):
            @pl.when(s == k)
            def _pub_hist():
                pltpu.sync_copy(hist_v, slots_sh[k])
        plsc.subcore_barrier()

        def pub(r_lo, sz):
            pltpu.sync_copy(summed_sh.at[pl.ds(r_lo, sz)],
                            summed_o.at[c, pl.ds(r_lo, sz)])
            for k in range(NS):
                pltpu.sync_copy(slots_sh[k].at[pl.ds(r_lo, sz)],
                                slab_v.at[pl.ds(k * stripe, sz)])
            for v in range(sz // 16):
                acc = slab_v[pl.ds(v * 16, 16)]
                for k in range(1, NS):
                    acc = acc + slab_v[pl.ds(k * stripe + v * 16, 16)]
                ridx = v * 16 + lax.iota(jnp.int32, 16)
                plsc.store_scatter(col_v, [ridx, jnp.zeros((16,), jnp.int32)],
                                   acc)
            pltpu.sync_copy(col_v.at[pl.ds(0, sz)],
                            cnt_o.at[c, pl.ds(r_lo, sz)])

        @pl.when(s < NS - 1)
        def _pub_full():
            pub(r0, stripe)

        @pl.when(s == NS - 1)
        def _pub_tail():
            pub((NS - 1) * stripe, s_tail)

        @pl.when(wid < NW - 1)
        def _full():
            b = pl.multiple_of(wid * 128, 128)
            pltpu.sync_copy(batch_h.at[pl.ds(b, 128)], src_v)
            pltpu.async_copy(attr_h.at[src_v], rows_v, sem).wait()
            pltpu.sync_copy(rows_v, attr_o.at[pl.ds(b, 128)])

        @pl.when(wid == NW - 1)
        def _tail():
            b = (NW - 1) * 128
            pltpu.sync_copy(batch_h.at[pl.ds(b, a_tail)],
                            src_v.at[pl.ds(0, a_tail)])
            pltpu.async_copy(attr_h.at[src_v.at[pl.ds(0, a_tail)]],
                             rows_v.at[pl.ds(0, a_tail)], sem).wait()
            pltpu.sync_copy(rows_v.at[pl.ds(0, a_tail)],
                            attr_o.at[pl.ds(b, a_tail)])

    return body(x, src, dst, batch, attr, zs)


def _tc_combine(summed2, cnt2, x_t, attr_g, W_l, W_r, W_lin, b_l, b_lin):
    n_dst, d = x_t.shape
    blk = 1000
    grid = n_dst // blk
    dn = (((1,), (1,)), ((), ()))

    def body(s2, c2, xt, ag, wl, wr, wlin, bl, blin, o):
        ssum = s2[0] + s2[1]
        cnt = c2[0] + c2[1]
        mean = ssum / jnp.maximum(cnt[:, 0:1], 1.0)
        acc = lax.dot_general(mean, wl[...], dn,
                              preferred_element_type=jnp.float32)
        acc = acc + lax.dot_general(xt[...], wr[...], dn,
                                    preferred_element_type=jnp.float32)
        acc = acc + 0.25 * lax.dot_general(ag[...], wlin[...], dn,
                                           preferred_element_type=jnp.float32)
        acc = acc + (bl[...] + 0.25 * blin[...])
        o[...] = jnp.maximum(acc, 0.0)

    return pl.pallas_call(
        body,
        grid=(grid,),
        in_specs=[
            pl.BlockSpec((NC, blk, d), lambda i: (0, i, 0)),
            pl.BlockSpec((NC, blk, 8), lambda i: (0, i, 0)),
            pl.BlockSpec((blk, d), lambda i: (i, 0)),
            pl.BlockSpec((blk, d), lambda i: (i, 0)),
            pl.BlockSpec((d, d), lambda i: (0, 0)),
            pl.BlockSpec((d, d), lambda i: (0, 0)),
            pl.BlockSpec((d, d), lambda i: (0, 0)),
            pl.BlockSpec((1, d), lambda i: (0, 0)),
            pl.BlockSpec((1, d), lambda i: (0, 0)),
        ],
        out_specs=pl.BlockSpec((blk, d), lambda i: (i, 0)),
        out_shape=jax.ShapeDtypeStruct((n_dst, d), jnp.float32),
    )(summed2, cnt2, x_t, attr_g, W_l, W_r, W_lin, b_l, b_lin)


def kernel(x, edge_index, batch, attr, W_l, b_l, W_r, W_lin, b_lin,
           size_src, size_dst):
    src = edge_index[0]
    dst = edge_index[1]
    n_dst = batch.shape[0]
    zs = jnp.zeros((n_dst, x.shape[1]), jnp.float32)
    summed2, cnt2, attr_g = _sc_agg(x, src, dst, batch, attr, zs)
    return _tc_combine(summed2, cnt2, x[:n_dst], attr_g, W_l, W_r, W_lin,
                       b_l.reshape(1, -1), b_lin.reshape(1, -1))


# speedup vs baseline: 4.4373x; 1.0393x over previous
import functools

import jax
import jax.numpy as jnp
from jax import lax
from jax.experimental import pallas as pl
from jax.experimental.pallas import tpu as pltpu
from jax.experimental.pallas import tpu_sc as plsc

NC = 2
NS = 16
NW = NC * NS
CHUNK = 128


def _sc_agg(x, il, batch, attr, zs):
    n_src, d = x.shape
    e = il.shape[0] // 2
    n_dst = zs.shape[0]
    n_chunks = e // CHUNK
    stripe = 256
    s_tail = n_dst - (NS - 1) * stripe
    a_tail = n_dst - (NW - 1) * 128

    mesh = plsc.VectorSubcoreMesh(core_axis_name="c", subcore_axis_name="s")

    @functools.partial(
        pl.kernel,
        out_type=(
            jax.ShapeDtypeStruct((NC, n_dst, d), jnp.float32),
            jax.ShapeDtypeStruct((NC, n_dst, 8), jnp.float32),
            jax.ShapeDtypeStruct((n_dst, d), jnp.float32),
        ),
        mesh=mesh,
        scratch_types=(
            pltpu.VMEM((2 * CHUNK,), jnp.int32),
            pltpu.VMEM((2 * CHUNK,), jnp.int32),
            pltpu.VMEM((2 * CHUNK,), jnp.int32),
            pltpu.VMEM((CHUNK,), jnp.int32),
            pltpu.VMEM((CHUNK,), jnp.int32),
            pltpu.VMEM((CHUNK,), jnp.int32),
            pltpu.VMEM((CHUNK, d), jnp.float32),
            pltpu.VMEM((CHUNK, d), jnp.float32),
            pltpu.VMEM((CHUNK, d), jnp.float32),
            pltpu.VMEM((n_dst,), jnp.float32),
            pltpu.VMEM((NS * stripe,), jnp.float32),
            pltpu.VMEM((stripe, 8), jnp.float32),
            pltpu.SemaphoreType.DMA,
            pltpu.SemaphoreType.DMA,
            pltpu.SemaphoreType.DMA,
            pltpu.SemaphoreType.DMA,
            pltpu.SemaphoreType.DMA,
            pltpu.SemaphoreType.DMA,
            pltpu.VMEM_SHARED((n_dst, d), jnp.float32),
        ) + tuple(pltpu.VMEM_SHARED((n_dst,), jnp.float32)
                  for _ in range(NS)),
        compiler_params=pltpu.CompilerParams(needs_layout_passes=False),
    )
    def body(x_h, il_h, batch_h, attr_h, zs_h,
             summed_o, cnt_o, attr_o,
             src_a, src_b, src_c, dst_a, dst_b, dst_c,
             rows_a, rows_b, rows_c,
             hist_v, slab_v, col_v, g_a, g_b, g_c, s_a, s_b, s_c,
             summed_sh, *slots_sh):
        bidx_v, rows_v, sem = dst_a, rows_a, g_a
        srcs = (src_a, src_b, src_c)
        dsts = (dst_a, dst_b, dst_c)
        rows2 = (rows_a, rows_b, rows_c)
        gsems = (g_a, g_b, g_c)
        ssems = (s_a, s_b, s_c)
        c = lax.axis_index("c")
        s = lax.axis_index("s")
        wid = s * NC + c

        r0 = pl.multiple_of(s * stripe, stripe)

        @pl.when(s < NS - 1)
        def _zero_full():
            pltpu.sync_copy(zs_h.at[pl.ds(r0, stripe)],
                            summed_sh.at[pl.ds(r0, stripe)])

        @pl.when(s == NS - 1)
        def _zero_tail():
            t0 = (NS - 1) * stripe
            pltpu.sync_copy(zs_h.at[pl.ds(t0, s_tail)],
                            summed_sh.at[pl.ds(t0, s_tail)])

        z16 = jnp.zeros((16,), jnp.float32)

        def zero_body(i, carry):
            hist_v[pl.ds(pl.multiple_of(i * 16, 16), 16)] = z16
            return carry

        lax.fori_loop(0, n_dst // 16, zero_body, 0)
        plsc.subcore_barrier()

        c0 = (n_chunks * wid) // NW
        c1 = (n_chunks * (wid + 1)) // NW
        one16 = jnp.ones((16,), jnp.float32)

        def load_and_fire(i, b):
            base = pl.multiple_of(i * 2 * CHUNK, 2 * CHUNK)
            pltpu.sync_copy(il_h.at[pl.ds(base, 2 * CHUNK)], srcs[b])
            pltpu.async_copy(x_h.at[srcs[b].at[pl.ds(0, CHUNK)]],
                             rows2[b], gsems[b])

        def wait_scatter(b):
            pltpu.make_async_copy(rows2[b], summed_sh.at[dsts[b]],
                                  ssems[b]).wait()

        load_and_fire(c0, 0)
        load_and_fire(c0 + 1, 1)

        def chunk_body(i, carry):
            for b in (0, 1, 2):
                @pl.when((i - c0) % 3 == b)
                def _():
                    pltpu.make_async_copy(x_h.at[srcs[b].at[pl.ds(0, CHUNK)]],
                                          rows2[b], gsems[b]).wait()
                    for j in range(CHUNK // 16):
                        dv = srcs[b][pl.ds(CHUNK + j * 16, 16)]
                        plsc.addupdate_scatter(hist_v, [dv], one16)
                        dsts[b][pl.ds(j * 16, 16)] = dv
                    pltpu.async_copy(rows2[b], summed_sh.at[dsts[b]],
                                     ssems[b], add=True)
                    bn = (b + 2) % 3

                    @pl.when(i > c0)
                    def _drain():
                        wait_scatter(bn)

                    @pl.when(i + 2 < c1)
                    def _pref():
                        load_and_fire(i + 2, bn)
            return carry

        lax.fori_loop(c0, c1, chunk_body, 0)
        for b in (0, 1, 2):
            @pl.when((c1 - 1 - c0) % 3 == b)
            def _final_drain():
                wait_scatter(b)
        for k in range(NS):
            @pl.when(s == k)
            def _pub_hist():
                pltpu.sync_copy(hist_v, slots_sh[k])
        plsc.subcore_barrier()

        def pub(r_lo, sz):
            pltpu.sync_copy(summed_sh.at[pl.ds(r_lo, sz)],
                            summed_o.at[c, pl.ds(r_lo, sz)])
            for k in range(NS):
                pltpu.sync_copy(slots_sh[k].at[pl.ds(r_lo, sz)],
                                slab_v.at[pl.ds(k * stripe, sz)])
            for v in range(sz // 16):
                acc = slab_v[pl.ds(v * 16, 16)]
                for k in range(1, NS):
                    acc = acc + slab_v[pl.ds(k * stripe + v * 16, 16)]
                ridx = v * 16 + lax.iota(jnp.int32, 16)
                plsc.store_scatter(col_v, [ridx, jnp.zeros((16,), jnp.int32)],
                                   acc)
            pltpu.sync_copy(col_v.at[pl.ds(0, sz)],
                            cnt_o.at[c, pl.ds(r_lo, sz)])

        @pl.when(s < NS - 1)
        def _pub_full():
            pub(r0, stripe)

        @pl.when(s == NS - 1)
        def _pub_tail():
            pub((NS - 1) * stripe, s_tail)

        @pl.when(wid < NW - 1)
        def _full():
            b = pl.multiple_of(wid * 128, 128)
            pltpu.sync_copy(batch_h.at[pl.ds(b, 128)], bidx_v)
            pltpu.async_copy(attr_h.at[bidx_v], rows_v, sem).wait()
            pltpu.sync_copy(rows_v, attr_o.at[pl.ds(b, 128)])

        @pl.when(wid == NW - 1)
        def _tail():
            b = (NW - 1) * 128
            pltpu.sync_copy(batch_h.at[pl.ds(b, a_tail)],
                            bidx_v.at[pl.ds(0, a_tail)])
            pltpu.async_copy(attr_h.at[bidx_v.at[pl.ds(0, a_tail)]],
                             rows_v.at[pl.ds(0, a_tail)], sem).wait()
            pltpu.sync_copy(rows_v.at[pl.ds(0, a_tail)],
                            attr_o.at[pl.ds(b, a_tail)])

    return body(x, il, batch, attr, zs)


def _tc_combine(summed2, cnt2, x_t, attr_g, W_l, W_r, W_lin, b_l, b_lin):
    n_dst, d = x_t.shape
    blk = 1000
    grid = n_dst // blk
    dn = (((1,), (1,)), ((), ()))

    def body(s2, c2, xt, ag, wl, wr, wlin, bl, blin, o):
        ssum = s2[0] + s2[1]
        cnt = c2[0] + c2[1]
        mean = ssum / jnp.maximum(cnt[:, 0:1], 1.0)
        acc = lax.dot_general(mean, wl[...], dn,
                              preferred_element_type=jnp.float32)
        acc = acc + lax.dot_general(xt[...], wr[...], dn,
                                    preferred_element_type=jnp.float32)
        acc = acc + 0.25 * lax.dot_general(ag[...], wlin[...], dn,
                                           preferred_element_type=jnp.float32)
        acc = acc + (bl[...] + 0.25 * blin[...])
        o[...] = jnp.maximum(acc, 0.0)

    return pl.pallas_call(
        body,
        grid=(grid,),
        in_specs=[
            pl.BlockSpec((NC, blk, d), lambda i: (0, i, 0)),
            pl.BlockSpec((NC, blk, 8), lambda i: (0, i, 0)),
            pl.BlockSpec((blk, d), lambda i: (i, 0)),
            pl.BlockSpec((blk, d), lambda i: (i, 0)),
            pl.BlockSpec((d, d), lambda i: (0, 0)),
            pl.BlockSpec((d, d), lambda i: (0, 0)),
            pl.BlockSpec((d, d), lambda i: (0, 0)),
            pl.BlockSpec((1, d), lambda i: (0, 0)),
            pl.BlockSpec((1, d), lambda i: (0, 0)),
        ],
        out_specs=pl.BlockSpec((blk, d), lambda i: (i, 0)),
        out_shape=jax.ShapeDtypeStruct((n_dst, d), jnp.float32),
    )(summed2, cnt2, x_t, attr_g, W_l, W_r, W_lin, b_l, b_lin)


def kernel(x, edge_index, batch, attr, W_l, b_l, W_r, W_lin, b_lin,
           size_src, size_dst):
    src = edge_index[0]
    dst = edge_index[1]
    n_dst = batch.shape[0]
    il = jnp.stack([src.reshape(-1, CHUNK), dst.reshape(-1, CHUNK)],
                   axis=1).reshape(-1)
    zs = jnp.zeros((n_dst, x.shape[1]), jnp.float32)
    summed2, cnt2, attr_g = _sc_agg(x, il, batch, attr, zs)
    return _tc_combine(summed2, cnt2, x[:n_dst], attr_g, W_l, W_r, W_lin,
                       b_l.reshape(1, -1), b_lin.reshape(1, -1))


# speedup vs baseline: 4.6517x; 1.0483x over previous
import functools

import jax
import jax.numpy as jnp
from jax import lax
from jax.experimental import pallas as pl
from jax.experimental.pallas import tpu as pltpu
from jax.experimental.pallas import tpu_sc as plsc

NC = 2
NS = 16
NW = NC * NS
CHUNK = 128


def _sc_agg(x, il, batch, attr, zs):
    n_src, d = x.shape
    e = il.shape[0] // 2
    n_dst = zs.shape[0]
    n_chunks = e // CHUNK
    stripe = 256
    s_tail = n_dst - (NS - 1) * stripe
    a_tail = n_dst - (NW - 1) * 128

    mesh = plsc.VectorSubcoreMesh(core_axis_name="c", subcore_axis_name="s")

    @functools.partial(
        pl.kernel,
        out_type=(
            jax.ShapeDtypeStruct((NC, n_dst, d), jnp.float32),
            jax.ShapeDtypeStruct((NC, n_dst, 8), jnp.float32),
            jax.ShapeDtypeStruct((n_dst, d), jnp.float32),
        ),
        mesh=mesh,
        scratch_types=(
            pltpu.VMEM((2 * CHUNK,), jnp.int32),
            pltpu.VMEM((2 * CHUNK,), jnp.int32),
            pltpu.VMEM((2 * CHUNK,), jnp.int32),
            pltpu.VMEM((CHUNK,), jnp.int32),
            pltpu.VMEM((CHUNK,), jnp.int32),
            pltpu.VMEM((CHUNK,), jnp.int32),
            pltpu.VMEM((CHUNK, d), jnp.float32),
            pltpu.VMEM((CHUNK, d), jnp.float32),
            pltpu.VMEM((CHUNK, d), jnp.float32),
            pltpu.VMEM((n_dst,), jnp.float32),
            pltpu.VMEM((NS * stripe,), jnp.float32),
            pltpu.VMEM((stripe, 8), jnp.float32),
            pltpu.SemaphoreType.DMA,
            pltpu.SemaphoreType.DMA,
            pltpu.SemaphoreType.DMA,
            pltpu.SemaphoreType.DMA,
            pltpu.SemaphoreType.DMA,
            pltpu.SemaphoreType.DMA,
            pltpu.SemaphoreType.DMA,
            pltpu.SemaphoreType.DMA,
            pltpu.SemaphoreType.DMA,
            pltpu.VMEM_SHARED((n_dst, d), jnp.float32),
        ) + tuple(pltpu.VMEM_SHARED((n_dst,), jnp.float32)
                  for _ in range(NS)),
        compiler_params=pltpu.CompilerParams(needs_layout_passes=False),
    )
    def body(x_h, il_h, batch_h, attr_h, zs_h,
             summed_o, cnt_o, attr_o,
             src_a, src_b, src_c, dst_a, dst_b, dst_c,
             rows_a, rows_b, rows_c,
             hist_v, slab_v, col_v, g_a, g_b, g_c, s_a, s_b, s_c,
             i_a, i_b, i_c,
             summed_sh, *slots_sh):
        bidx_v, rows_v, sem = dst_a, rows_a, g_a
        srcs = (src_a, src_b, src_c)
        dsts = (dst_a, dst_b, dst_c)
        rows2 = (rows_a, rows_b, rows_c)
        gsems = (g_a, g_b, g_c)
        ssems = (s_a, s_b, s_c)
        isems = (i_a, i_b, i_c)
        c = lax.axis_index("c")
        s = lax.axis_index("s")
        wid = s * NC + c

        r0 = pl.multiple_of(s * stripe, stripe)

        @pl.when(s < NS - 1)
        def _zero_full():
            pltpu.sync_copy(zs_h.at[pl.ds(r0, stripe)],
                            summed_sh.at[pl.ds(r0, stripe)])

        @pl.when(s == NS - 1)
        def _zero_tail():
            t0 = (NS - 1) * stripe
            pltpu.sync_copy(zs_h.at[pl.ds(t0, s_tail)],
                            summed_sh.at[pl.ds(t0, s_tail)])

        z16 = jnp.zeros((16,), jnp.float32)

        def zero_body(i, carry):
            hist_v[pl.ds(pl.multiple_of(i * 16, 16), 16)] = z16
            return carry

        lax.fori_loop(0, n_dst // 16, zero_body, 0)
        plsc.subcore_barrier()

        c0 = (n_chunks * wid) // NW
        c1 = (n_chunks * (wid + 1)) // NW
        one16 = jnp.ones((16,), jnp.float32)

        def il_slice(i):
            base = pl.multiple_of(i * 2 * CHUNK, 2 * CHUNK)
            return il_h.at[pl.ds(base, 2 * CHUNK)]

        def fire_idx(i, b):
            pltpu.async_copy(il_slice(i), srcs[b], isems[b])

        def wait_idx(i, b):
            pltpu.make_async_copy(il_slice(i), srcs[b], isems[b]).wait()

        def fire_gather(b):
            pltpu.async_copy(x_h.at[srcs[b].at[pl.ds(0, CHUNK)]],
                             rows2[b], gsems[b])

        def wait_scatter(b):
            pltpu.make_async_copy(rows2[b], summed_sh.at[dsts[b]],
                                  ssems[b]).wait()

        pltpu.sync_copy(il_slice(c0), src_a)
        pltpu.sync_copy(il_slice(c0 + 1), src_b)
        fire_gather(0)
        fire_gather(1)
        fire_idx(c0 + 2, 2)

        def chunk_body(i, carry):
            for b in (0, 1, 2):
                @pl.when((i - c0) % 3 == b)
                def _():
                    pltpu.make_async_copy(x_h.at[srcs[b].at[pl.ds(0, CHUNK)]],
                                          rows2[b], gsems[b]).wait()
                    for j in range(CHUNK // 16):
                        dv = srcs[b][pl.ds(CHUNK + j * 16, 16)]
                        plsc.addupdate_scatter(hist_v, [dv], one16)
                        dsts[b][pl.ds(j * 16, 16)] = dv
                    pltpu.async_copy(rows2[b], summed_sh.at[dsts[b]],
                                     ssems[b], add=True)
                    bn = (b + 2) % 3

                    @pl.when(i > c0)
                    def _drain():
                        wait_scatter(bn)

                    @pl.when(i + 2 < c1)
                    def _pref():
                        wait_idx(i + 2, bn)
                        fire_gather(bn)

                    @pl.when(i + 3 < c1)
                    def _pref_idx():
                        fire_idx(i + 3, b)
            return carry

        lax.fori_loop(c0, c1, chunk_body, 0)
        for b in (0, 1, 2):
            @pl.when((c1 - 1 - c0) % 3 == b)
            def _final_drain():
                wait_scatter(b)
        for k in range(NS):
            @pl.when(s == k)
            def _pub_hist():
                pltpu.sync_copy(hist_v, slots_sh[k])
        plsc.subcore_barrier()

        def pub(r_lo, sz):
            pltpu.sync_copy(summed_sh.at[pl.ds(r_lo, sz)],
                            summed_o.at[c, pl.ds(r_lo, sz)])
            for k in range(NS):
                pltpu.sync_copy(slots_sh[k].at[pl.ds(r_lo, sz)],
                                slab_v.at[pl.ds(k * stripe, sz)])
            for v in range(sz // 16):
                acc = slab_v[pl.ds(v * 16, 16)]
                for k in range(1, NS):
                    acc = acc + slab_v[pl.ds(k * stripe + v * 16, 16)]
                ridx = v * 16 + lax.iota(jnp.int32, 16)
                plsc.store_scatter(col_v, [ridx, jnp.zeros((16,), jnp.int32)],
                                   acc)
            pltpu.sync_copy(col_v.at[pl.ds(0, sz)],
                            cnt_o.at[c, pl.ds(r_lo, sz)])

        @pl.when(s < NS - 1)
        def _pub_full():
            pub(r0, stripe)

        @pl.when(s == NS - 1)
        def _pub_tail():
            pub((NS - 1) * stripe, s_tail)

        @pl.when(wid < NW - 1)
        def _full():
            b = pl.multiple_of(wid * 128, 128)
            pltpu.sync_copy(batch_h.at[pl.ds(b, 128)], bidx_v)
            pltpu.async_copy(attr_h.at[bidx_v], rows_v, sem).wait()
            pltpu.sync_copy(rows_v, attr_o.at[pl.ds(b, 128)])

        @pl.when(wid == NW - 1)
        def _tail():
            b = (NW - 1) * 128
            pltpu.sync_copy(batch_h.at[pl.ds(b, a_tail)],
                            bidx_v.at[pl.ds(0, a_tail)])
            pltpu.async_copy(attr_h.at[bidx_v.at[pl.ds(0, a_tail)]],
                             rows_v.at[pl.ds(0, a_tail)], sem).wait()
            pltpu.sync_copy(rows_v.at[pl.ds(0, a_tail)],
                            attr_o.at[pl.ds(b, a_tail)])

    return body(x, il, batch, attr, zs)


def _tc_combine(summed2, cnt2, x_t, attr_g, W_l, W_r, W_lin, b_l, b_lin):
    n_dst, d = x_t.shape
    blk = 1000
    grid = n_dst // blk
    dn = (((1,), (1,)), ((), ()))

    def body(s2, c2, xt, ag, wl, wr, wlin, bl, blin, o):
        ssum = s2[0] + s2[1]
        cnt = c2[0] + c2[1]
        mean = ssum / jnp.maximum(cnt[:, 0:1], 1.0)
        acc = lax.dot_general(mean, wl[...], dn,
                              preferred_element_type=jnp.float32)
        acc = acc + lax.dot_general(xt[...], wr[...], dn,
                                    preferred_element_type=jnp.float32)
        acc = acc + 0.25 * lax.dot_general(ag[...], wlin[...], dn,
                                           preferred_element_type=jnp.float32)
        acc = acc + (bl[...] + 0.25 * blin[...])
        o[...] = jnp.maximum(acc, 0.0)

    return pl.pallas_call(
        body,
        grid=(grid,),
        in_specs=[
            pl.BlockSpec((NC, blk, d), lambda i: (0, i, 0)),
            pl.BlockSpec((NC, blk, 8), lambda i: (0, i, 0)),
            pl.BlockSpec((blk, d), lambda i: (i, 0)),
            pl.BlockSpec((blk, d), lambda i: (i, 0)),
            pl.BlockSpec((d, d), lambda i: (0, 0)),
            pl.BlockSpec((d, d), lambda i: (0, 0)),
            pl.BlockSpec((d, d), lambda i: (0, 0)),
            pl.BlockSpec((1, d), lambda i: (0, 0)),
            pl.BlockSpec((1, d), lambda i: (0, 0)),
        ],
        out_specs=pl.BlockSpec((blk, d), lambda i: (i, 0)),
        out_shape=jax.ShapeDtypeStruct((n_dst, d), jnp.float32),
    )(summed2, cnt2, x_t, attr_g, W_l, W_r, W_lin, b_l, b_lin)


def kernel(x, edge_index, batch, attr, W_l, b_l, W_r, W_lin, b_lin,
           size_src, size_dst):
    src = edge_index[0]
    dst = edge_index[1]
    n_dst = batch.shape[0]
    il = jnp.stack([src.reshape(-1, CHUNK), dst.reshape(-1, CHUNK)],
                   axis=1).reshape(-1)
    zs = jnp.zeros((n_dst, x.shape[1]), jnp.float32)
    summed2, cnt2, attr_g = _sc_agg(x, il, batch, attr, zs)
    return _tc_combine(summed2, cnt2, x[:n_dst], attr_g, W_l, W_r, W_lin,
                       b_l.reshape(1, -1), b_lin.reshape(1, -1))


# speedup vs baseline: 4.7261x; 1.0160x over previous
import functools

import jax
import jax.numpy as jnp
from jax import lax
from jax.experimental import pallas as pl
from jax.experimental.pallas import tpu as pltpu
from jax.experimental.pallas import tpu_sc as plsc

NC = 2
NS = 16
NW = NC * NS
CHUNK = 128


def _sc_agg(x, il, batch, attr, n_dst):
    n_src, d = x.shape
    e = il.shape[0] // 2
    n_chunks = e // CHUNK
    stripe = 256
    s_tail = n_dst - (NS - 1) * stripe
    a_tail = n_dst - (NW - 1) * 128

    mesh = plsc.VectorSubcoreMesh(core_axis_name="c", subcore_axis_name="s")

    @functools.partial(
        pl.kernel,
        out_type=(
            jax.ShapeDtypeStruct((NC, n_dst, d), jnp.float32),
            jax.ShapeDtypeStruct((NC, n_dst, 8), jnp.float32),
            jax.ShapeDtypeStruct((n_dst, d), jnp.float32),
        ),
        mesh=mesh,
        scratch_types=(
            pltpu.VMEM((2 * CHUNK,), jnp.int32),
            pltpu.VMEM((2 * CHUNK,), jnp.int32),
            pltpu.VMEM((2 * CHUNK,), jnp.int32),
            pltpu.VMEM((CHUNK,), jnp.int32),
            pltpu.VMEM((CHUNK,), jnp.int32),
            pltpu.VMEM((CHUNK,), jnp.int32),
            pltpu.VMEM((CHUNK, d), jnp.float32),
            pltpu.VMEM((CHUNK, d), jnp.float32),
            pltpu.VMEM((CHUNK, d), jnp.float32),
            pltpu.VMEM((n_dst,), jnp.float32),
            pltpu.VMEM((NS * stripe,), jnp.float32),
            pltpu.VMEM((stripe, 8), jnp.float32),
            pltpu.SemaphoreType.DMA,
            pltpu.SemaphoreType.DMA,
            pltpu.SemaphoreType.DMA,
            pltpu.SemaphoreType.DMA,
            pltpu.SemaphoreType.DMA,
            pltpu.SemaphoreType.DMA,
            pltpu.SemaphoreType.DMA,
            pltpu.SemaphoreType.DMA,
            pltpu.SemaphoreType.DMA,
            pltpu.VMEM_SHARED((n_dst, d), jnp.float32),
        ) + tuple(pltpu.VMEM_SHARED((n_dst,), jnp.float32)
                  for _ in range(NS)),
        compiler_params=pltpu.CompilerParams(needs_layout_passes=False),
    )
    def body(x_h, il_h, batch_h, attr_h,
             summed_o, cnt_o, attr_o,
             src_a, src_b, src_c, dst_a, dst_b, dst_c,
             rows_a, rows_b, rows_c,
             hist_v, slab_v, col_v, g_a, g_b, g_c, s_a, s_b, s_c,
             i_a, i_b, i_c,
             summed_sh, *slots_sh):
        bidx_v, rows_v, sem = dst_a, rows_a, g_a
        srcs = (src_a, src_b, src_c)
        dsts = (dst_a, dst_b, dst_c)
        rows2 = (rows_a, rows_b, rows_c)
        gsems = (g_a, g_b, g_c)
        ssems = (s_a, s_b, s_c)
        isems = (i_a, i_b, i_c)
        c = lax.axis_index("c")
        s = lax.axis_index("s")
        wid = s * NC + c

        r0 = pl.multiple_of(s * stripe, stripe)
        z16 = jnp.zeros((16,), jnp.float32)

        def zrow_body(i, carry):
            rows_a[i, pl.ds(0, 16)] = z16
            for jq in range(1, d // 16):
                rows_a[i, pl.ds(pl.multiple_of(jq * 16, 16), 16)] = z16
            return carry

        lax.fori_loop(0, CHUNK, zrow_body, 0)

        @pl.when(s < NS - 1)
        def _zero_full():
            for q in range(stripe // CHUNK):
                pltpu.sync_copy(rows_a,
                                summed_sh.at[pl.ds(r0 + q * CHUNK, CHUNK)])

        @pl.when(s == NS - 1)
        def _zero_tail():
            t0 = (NS - 1) * stripe
            pltpu.sync_copy(rows_a, summed_sh.at[pl.ds(t0, CHUNK)])
            pltpu.sync_copy(rows_a.at[pl.ds(0, s_tail - CHUNK)],
                            summed_sh.at[pl.ds(t0 + CHUNK, s_tail - CHUNK)])

        def zero_body(i, carry):
            hist_v[pl.ds(pl.multiple_of(i * 16, 16), 16)] = z16
            return carry

        lax.fori_loop(0, n_dst // 16, zero_body, 0)
        plsc.subcore_barrier()

        c0 = (n_chunks * wid) // NW
        c1 = (n_chunks * (wid + 1)) // NW
        one16 = jnp.ones((16,), jnp.float32)

        def il_slice(i):
            base = pl.multiple_of(i * 2 * CHUNK, 2 * CHUNK)
            return il_h.at[pl.ds(base, 2 * CHUNK)]

        def fire_idx(i, b):
            pltpu.async_copy(il_slice(i), srcs[b], isems[b])

        def wait_idx(i, b):
            pltpu.make_async_copy(il_slice(i), srcs[b], isems[b]).wait()

        def fire_gather(b):
            pltpu.async_copy(x_h.at[srcs[b].at[pl.ds(0, CHUNK)]],
                             rows2[b], gsems[b])

        def wait_scatter(b):
            pltpu.make_async_copy(rows2[b], summed_sh.at[dsts[b]],
                                  ssems[b]).wait()

        pltpu.sync_copy(il_slice(c0), src_a)
        pltpu.sync_copy(il_slice(c0 + 1), src_b)
        fire_gather(0)
        fire_gather(1)
        fire_idx(c0 + 2, 2)

        def chunk_body(i, carry):
            for b in (0, 1, 2):
                @pl.when((i - c0) % 3 == b)
                def _():
                    pltpu.make_async_copy(x_h.at[srcs[b].at[pl.ds(0, CHUNK)]],
                                          rows2[b], gsems[b]).wait()
                    for j in range(CHUNK // 16):
                        dv = srcs[b][pl.ds(CHUNK + j * 16, 16)]
                        plsc.addupdate_scatter(hist_v, [dv], one16)
                        dsts[b][pl.ds(j * 16, 16)] = dv
                    pltpu.async_copy(rows2[b], summed_sh.at[dsts[b]],
                                     ssems[b], add=True)
                    bn = (b + 2) % 3

                    @pl.when(i > c0)
                    def _drain():
                        wait_scatter(bn)

                    @pl.when(i + 2 < c1)
                    def _pref():
                        wait_idx(i + 2, bn)
                        fire_gather(bn)

                    @pl.when(i + 3 < c1)
                    def _pref_idx():
                        fire_idx(i + 3, b)
            return carry

        lax.fori_loop(c0, c1, chunk_body, 0)
        for b in (0, 1, 2):
            @pl.when((c1 - 1 - c0) % 3 == b)
            def _final_drain():
                wait_scatter(b)
        for k in range(NS):
            @pl.when(s == k)
            def _pub_hist():
                pltpu.sync_copy(hist_v, slots_sh[k])
        plsc.subcore_barrier()

        def pub(r_lo, sz):
            pltpu.sync_copy(summed_sh.at[pl.ds(r_lo, sz)],
                            summed_o.at[c, pl.ds(r_lo, sz)])
            for k in range(NS):
                pltpu.sync_copy(slots_sh[k].at[pl.ds(r_lo, sz)],
                                slab_v.at[pl.ds(k * stripe, sz)])
            for v in range(sz // 16):
                acc = slab_v[pl.ds(v * 16, 16)]
                for k in range(1, NS):
                    acc = acc + slab_v[pl.ds(k * stripe + v * 16, 16)]
                ridx = v * 16 + lax.iota(jnp.int32, 16)
                plsc.store_scatter(col_v, [ridx, jnp.zeros((16,), jnp.int32)],
                                   acc)
            pltpu.sync_copy(col_v.at[pl.ds(0, sz)],
                            cnt_o.at[c, pl.ds(r_lo, sz)])

        @pl.when(s < NS - 1)
        def _pub_full():
            pub(r0, stripe)

        @pl.when(s == NS - 1)
        def _pub_tail():
            pub((NS - 1) * stripe, s_tail)

        @pl.when(wid < NW - 1)
        def _full():
            b = pl.multiple_of(wid * 128, 128)
            pltpu.sync_copy(batch_h.at[pl.ds(b, 128)], bidx_v)
            pltpu.async_copy(attr_h.at[bidx_v], rows_v, sem).wait()
            pltpu.sync_copy(rows_v, attr_o.at[pl.ds(b, 128)])

        @pl.when(wid == NW - 1)
        def _tail():
            b = (NW - 1) * 128
            pltpu.sync_copy(batch_h.at[pl.ds(b, a_tail)],
                            bidx_v.at[pl.ds(0, a_tail)])
            pltpu.async_copy(attr_h.at[bidx_v.at[pl.ds(0, a_tail)]],
                             rows_v.at[pl.ds(0, a_tail)], sem).wait()
            pltpu.sync_copy(rows_v.at[pl.ds(0, a_tail)],
                            attr_o.at[pl.ds(b, a_tail)])

    return body(x, il, batch, attr)


def _tc_combine(summed2, cnt2, x_full, n_dst, attr_g, W_l, W_r, W_lin,
                b_l, b_lin):
    d = x_full.shape[1]
    blk = 1000
    grid = n_dst // blk
    dn = (((1,), (1,)), ((), ()))

    def body(s2, c2, xt, ag, wl, wr, wlin, bl, blin, o):
        ssum = s2[0] + s2[1]
        cnt = c2[0] + c2[1]
        mean = ssum / jnp.maximum(cnt[:, 0:1], 1.0)
        acc = lax.dot_general(mean, wl[...], dn,
                              preferred_element_type=jnp.float32)
        acc = acc + lax.dot_general(xt[...], wr[...], dn,
                                    preferred_element_type=jnp.float32)
        acc = acc + 0.25 * lax.dot_general(ag[...], wlin[...], dn,
                                           preferred_element_type=jnp.float32)
        acc = acc + (bl[...] + 0.25 * blin[...])
        o[...] = jnp.maximum(acc, 0.0)

    return pl.pallas_call(
        body,
        grid=(grid,),
        in_specs=[
            pl.BlockSpec((NC, blk, d), lambda i: (0, i, 0)),
            pl.BlockSpec((NC, blk, 8), lambda i: (0, i, 0)),
            pl.BlockSpec((blk, d), lambda i: (i, 0)),
            pl.BlockSpec((blk, d), lambda i: (i, 0)),
            pl.BlockSpec((d, d), lambda i: (0, 0)),
            pl.BlockSpec((d, d), lambda i: (0, 0)),
            pl.BlockSpec((d, d), lambda i: (0, 0)),
            pl.BlockSpec((1, d), lambda i: (0, 0)),
            pl.BlockSpec((1, d), lambda i: (0, 0)),
        ],
        out_specs=pl.BlockSpec((blk, d), lambda i: (i, 0)),
        out_shape=jax.ShapeDtypeStruct((n_dst, d), jnp.float32),
    )(summed2, cnt2, x_full, attr_g, W_l, W_r, W_lin, b_l, b_lin)


def kernel(x, edge_index, batch, attr, W_l, b_l, W_r, W_lin, b_lin,
           size_src, size_dst):
    src = edge_index[0]
    dst = edge_index[1]
    n_dst = batch.shape[0]
    il = jnp.stack([src.reshape(-1, CHUNK), dst.reshape(-1, CHUNK)],
                   axis=1).reshape(-1)
    summed2, cnt2, attr_g = _sc_agg(x, il, batch, attr, n_dst)
    return _tc_combine(summed2, cnt2, x, n_dst, attr_g, W_l, W_r, W_lin,
                       b_l.reshape(1, -1), b_lin.reshape(1, -1))


# speedup vs baseline: 4.7777x; 1.0109x over previous
import functools

import jax
import jax.numpy as jnp
from jax import lax
from jax.experimental import pallas as pl
from jax.experimental.pallas import tpu as pltpu
from jax.experimental.pallas import tpu_sc as plsc

NC = 2
NS = 16
NW = NC * NS
CHUNK = 128


def _sc_agg(x, il, batch, attr, n_dst):
    n_src, d = x.shape
    e = il.shape[0] // 2
    n_chunks = e // CHUNK
    stripe = 256
    s_tail = n_dst - (NS - 1) * stripe
    a_tail = n_dst - (NW - 1) * 128

    mesh = plsc.VectorSubcoreMesh(core_axis_name="c", subcore_axis_name="s")

    @functools.partial(
        pl.kernel,
        out_type=(
            jax.ShapeDtypeStruct((NC, n_dst, d), jnp.float32),
            jax.ShapeDtypeStruct((NC, n_dst, 8), jnp.float32),
            jax.ShapeDtypeStruct((n_dst, d), jnp.float32),
        ),
        mesh=mesh,
        scratch_types=(
            pltpu.VMEM((2 * CHUNK,), jnp.int32),
            pltpu.VMEM((2 * CHUNK,), jnp.int32),
            pltpu.VMEM((2 * CHUNK,), jnp.int32),
            pltpu.VMEM((CHUNK,), jnp.int32),
            pltpu.VMEM((CHUNK,), jnp.int32),
            pltpu.VMEM((CHUNK,), jnp.int32),
            pltpu.VMEM((CHUNK, d), jnp.float32),
            pltpu.VMEM((CHUNK, d), jnp.float32),
            pltpu.VMEM((CHUNK, d), jnp.float32),
            pltpu.VMEM((n_dst,), jnp.float32),
            pltpu.VMEM((NS * stripe,), jnp.float32),
            pltpu.VMEM((stripe, 8), jnp.float32),
            pltpu.SemaphoreType.DMA,
            pltpu.SemaphoreType.DMA,
            pltpu.SemaphoreType.DMA,
            pltpu.SemaphoreType.DMA,
            pltpu.SemaphoreType.DMA,
            pltpu.SemaphoreType.DMA,
            pltpu.SemaphoreType.DMA,
            pltpu.SemaphoreType.DMA,
            pltpu.SemaphoreType.DMA,
            pltpu.VMEM_SHARED((n_dst, d), jnp.float32),
        ) + tuple(pltpu.VMEM_SHARED((n_dst,), jnp.float32)
                  for _ in range(NS)),
        compiler_params=pltpu.CompilerParams(needs_layout_passes=False),
    )
    def body(x_h, il_h, batch_h, attr_h,
             summed_o, cnt_o, attr_o,
             src_a, src_b, src_c, dst_a, dst_b, dst_c,
             rows_a, rows_b, rows_c,
             hist_v, slab_v, col_v, g_a, g_b, g_c, s_a, s_b, s_c,
             i_a, i_b, i_c,
             summed_sh, *slots_sh):
        bidx_v, rows_v, sem = dst_a, rows_a, g_a
        srcs = (src_a, src_b, src_c)
        dsts = (dst_a, dst_b, dst_c)
        rows2 = (rows_a, rows_b, rows_c)
        gsems = (g_a, g_b, g_c)
        ssems = (s_a, s_b, s_c)
        isems = (i_a, i_b, i_c)
        c = lax.axis_index("c")
        s = lax.axis_index("s")
        wid = s * NC + c

        r0 = pl.multiple_of(s * stripe, stripe)
        z16 = jnp.zeros((16,), jnp.float32)

        def zrow_body(i, carry):
            rows_a[i, pl.ds(0, 16)] = z16
            for jq in range(1, d // 16):
                rows_a[i, pl.ds(pl.multiple_of(jq * 16, 16), 16)] = z16
            return carry

        lax.fori_loop(0, CHUNK, zrow_body, 0)

        @pl.when(s < NS - 1)
        def _zero_full():
            for q in range(stripe // CHUNK):
                pltpu.sync_copy(rows_a,
                                summed_sh.at[pl.ds(r0 + q * CHUNK, CHUNK)])

        @pl.when(s == NS - 1)
        def _zero_tail():
            t0 = (NS - 1) * stripe
            pltpu.sync_copy(rows_a, summed_sh.at[pl.ds(t0, CHUNK)])
            pltpu.sync_copy(rows_a.at[pl.ds(0, s_tail - CHUNK)],
                            summed_sh.at[pl.ds(t0 + CHUNK, s_tail - CHUNK)])

        def zero_body(i, carry):
            hist_v[pl.ds(pl.multiple_of(i * 16, 16), 16)] = z16
            return carry

        lax.fori_loop(0, n_dst // 16, zero_body, 0)
        plsc.subcore_barrier()

        c0 = (n_chunks * wid) // NW
        c1 = (n_chunks * (wid + 1)) // NW
        one16 = jnp.ones((16,), jnp.float32)

        def il_slice(i):
            base = pl.multiple_of(i * 2 * CHUNK, 2 * CHUNK)
            return il_h.at[pl.ds(base, 2 * CHUNK)]

        def fire_idx(i, b):
            pltpu.async_copy(il_slice(i), srcs[b], isems[b])

        def wait_idx(i, b):
            pltpu.make_async_copy(il_slice(i), srcs[b], isems[b]).wait()

        def fire_gather(b):
            pltpu.async_copy(x_h.at[srcs[b].at[pl.ds(0, CHUNK)]],
                             rows2[b], gsems[b])

        def wait_scatter(b):
            pltpu.make_async_copy(rows2[b], summed_sh.at[dsts[b]],
                                  ssems[b]).wait()

        pltpu.sync_copy(il_slice(c0), src_a)
        pltpu.sync_copy(il_slice(c0 + 1), src_b)
        fire_gather(0)
        fire_gather(1)
        fire_idx(c0 + 2, 2)

        def chunk_body(i, carry):
            for b in (0, 1, 2):
                @pl.when((i - c0) % 3 == b)
                def _():
                    pltpu.make_async_copy(x_h.at[srcs[b].at[pl.ds(0, CHUNK)]],
                                          rows2[b], gsems[b]).wait()
                    for j in range(CHUNK // 16):
                        dv = srcs[b][pl.ds(CHUNK + j * 16, 16)]
                        plsc.addupdate_scatter(hist_v, [dv], one16)
                        dsts[b][pl.ds(j * 16, 16)] = dv
                    pltpu.async_copy(rows2[b], summed_sh.at[dsts[b]],
                                     ssems[b], add=True)
                    bn = (b + 2) % 3

                    @pl.when(i > c0)
                    def _drain():
                        wait_scatter(bn)

                    @pl.when(i + 2 < c1)
                    def _pref():
                        wait_idx(i + 2, bn)
                        fire_gather(bn)

                    @pl.when(i + 3 < c1)
                    def _pref_idx():
                        fire_idx(i + 3, b)
            return carry

        lax.fori_loop(c0, c1, chunk_body, 0)
        for b in (0, 1, 2):
            @pl.when((c1 - 1 - c0) % 3 == b)
            def _final_drain():
                wait_scatter(b)
        for k in range(NS):
            @pl.when(s == k)
            def _pub_hist():
                pltpu.sync_copy(hist_v, slots_sh[k])
        plsc.subcore_barrier()

        def pub(r_lo, sz):
            pltpu.sync_copy(summed_sh.at[pl.ds(r_lo, sz)],
                            summed_o.at[c, pl.ds(r_lo, sz)])
            cps = [pltpu.async_copy(slots_sh[k].at[pl.ds(r_lo, sz)],
                                    slab_v.at[pl.ds(k * stripe, sz)], i_a)
                   for k in range(NS)]
            for cp in cps:
                cp.wait()
            for v in range(sz // 16):
                acc = slab_v[pl.ds(v * 16, 16)]
                for k in range(1, NS):
                    acc = acc + slab_v[pl.ds(k * stripe + v * 16, 16)]
                ridx = v * 16 + lax.iota(jnp.int32, 16)
                plsc.store_scatter(col_v, [ridx, jnp.zeros((16,), jnp.int32)],
                                   acc)
            pltpu.sync_copy(col_v.at[pl.ds(0, sz)],
                            cnt_o.at[c, pl.ds(r_lo, sz)])

        @pl.when(s < NS - 1)
        def _pub_full():
            pub(r0, stripe)

        @pl.when(s == NS - 1)
        def _pub_tail():
            pub((NS - 1) * stripe, s_tail)

        @pl.when(wid < NW - 1)
        def _full():
            b = pl.multiple_of(wid * 128, 128)
            pltpu.sync_copy(batch_h.at[pl.ds(b, 128)], bidx_v)
            pltpu.async_copy(attr_h.at[bidx_v], rows_v, sem).wait()
            pltpu.sync_copy(rows_v, attr_o.at[pl.ds(b, 128)])

        @pl.when(wid == NW - 1)
        def _tail():
            b = (NW - 1) * 128
            pltpu.sync_copy(batch_h.at[pl.ds(b, a_tail)],
                            bidx_v.at[pl.ds(0, a_tail)])
            pltpu.async_copy(attr_h.at[bidx_v.at[pl.ds(0, a_tail)]],
                             rows_v.at[pl.ds(0, a_tail)], sem).wait()
            pltpu.sync_copy(rows_v.at[pl.ds(0, a_tail)],
                            attr_o.at[pl.ds(b, a_tail)])

    return body(x, il, batch, attr)


def _tc_combine(summed2, cnt2, x_full, n_dst, attr_g, W_l, W_r, W_lin,
                b_l, b_lin):
    d = x_full.shape[1]
    blk = n_dst
    grid = n_dst // blk
    dn = (((1,), (1,)), ((), ()))

    def body(s2, c2, xt, ag, wl, wr, wlin, bl, blin, o):
        ssum = s2[0] + s2[1]
        cnt = c2[0] + c2[1]
        mean = ssum / jnp.maximum(cnt[:, 0:1], 1.0)
        acc = lax.dot_general(mean, wl[...], dn,
                              preferred_element_type=jnp.float32)
        acc = acc + lax.dot_general(xt[...], wr[...], dn,
                                    preferred_element_type=jnp.float32)
        acc = acc + 0.25 * lax.dot_general(ag[...], wlin[...], dn,
                                           preferred_element_type=jnp.float32)
        acc = acc + (bl[...] + 0.25 * blin[...])
        o[...] = jnp.maximum(acc, 0.0)

    return pl.pallas_call(
        body,
        grid=(grid,),
        in_specs=[
            pl.BlockSpec((NC, blk, d), lambda i: (0, i, 0)),
            pl.BlockSpec((NC, blk, 8), lambda i: (0, i, 0)),
            pl.BlockSpec((blk, d), lambda i: (i, 0)),
            pl.BlockSpec((blk, d), lambda i: (i, 0)),
            pl.BlockSpec((d, d), lambda i: (0, 0)),
            pl.BlockSpec((d, d), lambda i: (0, 0)),
            pl.BlockSpec((d, d), lambda i: (0, 0)),
            pl.BlockSpec((1, d), lambda i: (0, 0)),
            pl.BlockSpec((1, d), lambda i: (0, 0)),
        ],
        out_specs=pl.BlockSpec((blk, d), lambda i: (i, 0)),
        out_shape=jax.ShapeDtypeStruct((n_dst, d), jnp.float32),
    )(summed2, cnt2, x_full, attr_g, W_l, W_r, W_lin, b_l, b_lin)


def kernel(x, edge_index, batch, attr, W_l, b_l, W_r, W_lin, b_lin,
           size_src, size_dst):
    src = edge_index[0]
    dst = edge_index[1]
    n_dst = batch.shape[0]
    il = jnp.stack([src.reshape(-1, CHUNK), dst.reshape(-1, CHUNK)],
                   axis=1).reshape(-1)
    summed2, cnt2, attr_g = _sc_agg(x, il, batch, attr, n_dst)
    return _tc_combine(summed2, cnt2, x, n_dst, attr_g, W_l, W_r, W_lin,
                       b_l.reshape(1, -1), b_lin.reshape(1, -1))


# speedup vs baseline: 4.9130x; 1.0283x over previous
import functools

import jax
import jax.numpy as jnp
from jax import lax
from jax.experimental import pallas as pl
from jax.experimental.pallas import tpu as pltpu
from jax.experimental.pallas import tpu_sc as plsc

NC = 2
NS = 16
NW = NC * NS
CHUNK = 128


def _sc_agg(x, il, batch, attr, n_dst):
    n_src, d = x.shape
    e = il.shape[0] // 2
    n_chunks = e // CHUNK
    stripe = 256
    s_tail = n_dst - (NS - 1) * stripe
    a_tail = n_dst - (NW - 1) * 128

    mesh = plsc.VectorSubcoreMesh(core_axis_name="c", subcore_axis_name="s")

    @functools.partial(
        pl.kernel,
        out_type=(
            jax.ShapeDtypeStruct((NC, n_dst, d), jnp.float32),
            jax.ShapeDtypeStruct((NC, n_dst, 8), jnp.float32),
            jax.ShapeDtypeStruct((n_dst, d), jnp.float32),
        ),
        mesh=mesh,
        scratch_types=(
            pltpu.VMEM((2 * CHUNK,), jnp.int32),
            pltpu.VMEM((2 * CHUNK,), jnp.int32),
            pltpu.VMEM((2 * CHUNK,), jnp.int32),
            pltpu.VMEM((CHUNK,), jnp.int32),
            pltpu.VMEM((CHUNK,), jnp.int32),
            pltpu.VMEM((CHUNK,), jnp.int32),
            pltpu.VMEM((CHUNK, d), jnp.float32),
            pltpu.VMEM((CHUNK, d), jnp.float32),
            pltpu.VMEM((CHUNK, d), jnp.float32),
            pltpu.VMEM((n_dst,), jnp.float32),
            pltpu.VMEM((NS * stripe,), jnp.float32),
            pltpu.VMEM((stripe, 8), jnp.float32),
            pltpu.SemaphoreType.DMA,
            pltpu.SemaphoreType.DMA,
            pltpu.SemaphoreType.DMA,
            pltpu.SemaphoreType.DMA,
            pltpu.SemaphoreType.DMA,
            pltpu.SemaphoreType.DMA,
            pltpu.SemaphoreType.DMA,
            pltpu.SemaphoreType.DMA,
            pltpu.SemaphoreType.DMA,
            pltpu.VMEM_SHARED((n_dst, d), jnp.float32),
        ) + tuple(pltpu.VMEM_SHARED((n_dst,), jnp.float32)
                  for _ in range(NS)),
        compiler_params=pltpu.CompilerParams(needs_layout_passes=False),
    )
    def body(x_h, il_h, batch_h, attr_h,
             summed_o, cnt_o, attr_o,
             src_a, src_b, src_c, dst_a, dst_b, dst_c,
             rows_a, rows_b, rows_c,
             hist_v, slab_v, col_v, g_a, g_b, g_c, s_a, s_b, s_c,
             i_a, i_b, i_c,
             summed_sh, *slots_sh):
        bidx_v, rows_v, sem = dst_a, rows_a, g_a
        srcs = (src_a, src_b, src_c)
        dsts = (dst_a, dst_b, dst_c)
        rows2 = (rows_a, rows_b, rows_c)
        gsems = (g_a, g_b, g_c)
        ssems = (s_a, s_b, s_c)
        isems = (i_a, i_b, i_c)
        c = lax.axis_index("c")
        s = lax.axis_index("s")
        wid = s * NC + c

        r0 = pl.multiple_of(s * stripe, stripe)
        z16 = jnp.zeros((16,), jnp.float32)

        def zrow_body(i, carry):
            rows_a[i, pl.ds(0, 16)] = z16
            for jq in range(1, d // 16):
                rows_a[i, pl.ds(pl.multiple_of(jq * 16, 16), 16)] = z16
            return carry

        lax.fori_loop(0, CHUNK, zrow_body, 0)

        @pl.when(s < NS - 1)
        def _zero_full():
            for q in range(stripe // CHUNK):
                pltpu.sync_copy(rows_a,
                                summed_sh.at[pl.ds(r0 + q * CHUNK, CHUNK)])

        @pl.when(s == NS - 1)
        def _zero_tail():
            t0 = (NS - 1) * stripe
            pltpu.sync_copy(rows_a, summed_sh.at[pl.ds(t0, CHUNK)])
            pltpu.sync_copy(rows_a.at[pl.ds(0, s_tail - CHUNK)],
                            summed_sh.at[pl.ds(t0 + CHUNK, s_tail - CHUNK)])

        def zero_body(i, carry):
            hist_v[pl.ds(pl.multiple_of(i * 16, 16), 16)] = z16
            return carry

        lax.fori_loop(0, n_dst // 16, zero_body, 0)
        plsc.subcore_barrier()

        c0 = (n_chunks * wid) // NW
        c1 = (n_chunks * (wid + 1)) // NW
        one16 = jnp.ones((16,), jnp.float32)

        def il_slice(i):
            base = pl.multiple_of(i * 2 * CHUNK, 2 * CHUNK)
            return il_h.at[pl.ds(base, 2 * CHUNK)]

        def fire_idx(i, b):
            pltpu.async_copy(il_slice(i), srcs[b], isems[b])

        def wait_idx(i, b):
            pltpu.make_async_copy(il_slice(i), srcs[b], isems[b]).wait()

        def fire_gather(b):
            pltpu.async_copy(x_h.at[srcs[b].at[pl.ds(0, CHUNK)]],
                             rows2[b], gsems[b])

        def wait_scatter(b):
            pltpu.make_async_copy(rows2[b], summed_sh.at[dsts[b]],
                                  ssems[b]).wait()

        pltpu.sync_copy(il_slice(c0), src_a)
        pltpu.sync_copy(il_slice(c0 + 1), src_b)
        fire_gather(0)
        fire_gather(1)
        fire_idx(c0 + 2, 2)

        def chunk_body(i, carry):
            for b in (0, 1, 2):
                @pl.when((i - c0) % 3 == b)
                def _():
                    pltpu.make_async_copy(x_h.at[srcs[b].at[pl.ds(0, CHUNK)]],
                                          rows2[b], gsems[b]).wait()
                    bn = (b + 2) % 3

                    @pl.when(i > c0)
                    def _drain():
                        wait_scatter(bn)

                    @pl.when(i + 2 < c1)
                    def _pref():
                        wait_idx(i + 2, bn)
                        fire_gather(bn)

                    for j in range(CHUNK // 16):
                        dv = srcs[b][pl.ds(CHUNK + j * 16, 16)]
                        plsc.addupdate_scatter(hist_v, [dv], one16)
                        dsts[b][pl.ds(j * 16, 16)] = dv
                    pltpu.async_copy(rows2[b], summed_sh.at[dsts[b]],
                                     ssems[b], add=True)

                    @pl.when(i + 3 < c1)
                    def _pref_idx():
                        fire_idx(i + 3, b)
            return carry

        lax.fori_loop(c0, c1, chunk_body, 0)
        for b in (0, 1, 2):
            @pl.when((c1 - 1 - c0) % 3 == b)
            def _final_drain():
                wait_scatter(b)
        for k in range(NS):
            @pl.when(s == k)
            def _pub_hist():
                pltpu.sync_copy(hist_v, slots_sh[k])
        plsc.subcore_barrier()

        def pub(r_lo, sz):
            pltpu.sync_copy(summed_sh.at[pl.ds(r_lo, sz)],
                            summed_o.at[c, pl.ds(r_lo, sz)])
            cps = [pltpu.async_copy(slots_sh[k].at[pl.ds(r_lo, sz)],
                                    slab_v.at[pl.ds(k * stripe, sz)], i_a)
                   for k in range(NS)]
            for cp in cps:
                cp.wait()
            for v in range(sz // 16):
                acc = slab_v[pl.ds(v * 16, 16)]
                for k in range(1, NS):
                    acc = acc + slab_v[pl.ds(k * stripe + v * 16, 16)]
                ridx = v * 16 + lax.iota(jnp.int32, 16)
                plsc.store_scatter(col_v, [ridx, jnp.zeros((16,), jnp.int32)],
                                   acc)
            pltpu.sync_copy(col_v.at[pl.ds(0, sz)],
                            cnt_o.at[c, pl.ds(r_lo, sz)])

        @pl.when(s < NS - 1)
        def _pub_full():
            pub(r0, stripe)

        @pl.when(s == NS - 1)
        def _pub_tail():
            pub((NS - 1) * stripe, s_tail)

        @pl.when(wid < NW - 1)
        def _full():
            b = pl.multiple_of(wid * 128, 128)
            pltpu.sync_copy(batch_h.at[pl.ds(b, 128)], bidx_v)
            pltpu.async_copy(attr_h.at[bidx_v], rows_v, sem).wait()
            pltpu.sync_copy(rows_v, attr_o.at[pl.ds(b, 128)])

        @pl.when(wid == NW - 1)
        def _tail():
            b = (NW - 1) * 128
            pltpu.sync_copy(batch_h.at[pl.ds(b, a_tail)],
                            bidx_v.at[pl.ds(0, a_tail)])
            pltpu.async_copy(attr_h.at[bidx_v.at[pl.ds(0, a_tail)]],
                             rows_v.at[pl.ds(0, a_tail)], sem).wait()
            pltpu.sync_copy(rows_v.at[pl.ds(0, a_tail)],
                            attr_o.at[pl.ds(b, a_tail)])

    return body(x, il, batch, attr)


def _tc_combine(summed2, cnt2, x_full, n_dst, attr_g, W_l, W_r, W_lin,
                b_l, b_lin):
    d = x_full.shape[1]
    blk = n_dst
    grid = n_dst // blk
    dn = (((1,), (1,)), ((), ()))

    def body(s2, c2, xt, ag, wl, wr, wlin, bl, blin, o):
        ssum = s2[0] + s2[1]
        cnt = c2[0] + c2[1]
        mean = ssum / jnp.maximum(cnt[:, 0:1], 1.0)
        acc = lax.dot_general(mean, wl[...], dn,
                              preferred_element_type=jnp.float32)
        acc = acc + lax.dot_general(xt[...], wr[...], dn,
                                    preferred_element_type=jnp.float32)
        acc = acc + 0.25 * lax.dot_general(ag[...], wlin[...], dn,
                                           preferred_element_type=jnp.float32)
        acc = acc + (bl[...] + 0.25 * blin[...])
        o[...] = jnp.maximum(acc, 0.0)

    return pl.pallas_call(
        body,
        grid=(grid,),
        in_specs=[
            pl.BlockSpec((NC, blk, d), lambda i: (0, i, 0)),
            pl.BlockSpec((NC, blk, 8), lambda i: (0, i, 0)),
            pl.BlockSpec((blk, d), lambda i: (i, 0)),
            pl.BlockSpec((blk, d), lambda i: (i, 0)),
            pl.BlockSpec((d, d), lambda i: (0, 0)),
            pl.BlockSpec((d, d), lambda i: (0, 0)),
            pl.BlockSpec((d, d), lambda i: (0, 0)),
            pl.BlockSpec((1, d), lambda i: (0, 0)),
            pl.BlockSpec((1, d), lambda i: (0, 0)),
        ],
        out_specs=pl.BlockSpec((blk, d), lambda i: (i, 0)),
        out_shape=jax.ShapeDtypeStruct((n_dst, d), jnp.float32),
    )(summed2, cnt2, x_full, attr_g, W_l, W_r, W_lin, b_l, b_lin)


def kernel(x, edge_index, batch, attr, W_l, b_l, W_r, W_lin, b_lin,
           size_src, size_dst):
    src = edge_index[0]
    dst = edge_index[1]
    n_dst = batch.shape[0]
    il = jnp.stack([src.reshape(-1, CHUNK), dst.reshape(-1, CHUNK)],
                   axis=1).reshape(-1)
    summed2, cnt2, attr_g = _sc_agg(x, il, batch, attr, n_dst)
    return _tc_combine(summed2, cnt2, x, n_dst, attr_g, W_l, W_r, W_lin,
                       b_l.reshape(1, -1), b_lin.reshape(1, -1))


# speedup vs baseline: 4.9304x; 1.0035x over previous
import functools

import jax
import jax.numpy as jnp
from jax import lax
from jax.experimental import pallas as pl
from jax.experimental.pallas import tpu as pltpu
from jax.experimental.pallas import tpu_sc as plsc

NC = 2
NS = 16
NW = NC * NS
CHUNK = 128


def _sc_agg(x, il, batch, attr, n_dst):
    n_src, d = x.shape
    e = il.shape[0] // 2
    n_chunks = e // CHUNK
    stripe = 256
    s_tail = n_dst - (NS - 1) * stripe
    a_tail = n_dst - (NW - 1) * 128

    mesh = plsc.VectorSubcoreMesh(core_axis_name="c", subcore_axis_name="s")

    @functools.partial(
        pl.kernel,
        out_type=(
            jax.ShapeDtypeStruct((NC, n_dst, d), jnp.float32),
            jax.ShapeDtypeStruct((NC, n_dst, 8), jnp.float32),
            jax.ShapeDtypeStruct((n_dst, d), jnp.float32),
        ),
        mesh=mesh,
        scratch_types=(
            pltpu.VMEM((2 * CHUNK,), jnp.int32),
            pltpu.VMEM((2 * CHUNK,), jnp.int32),
            pltpu.VMEM((2 * CHUNK,), jnp.int32),
            pltpu.VMEM((CHUNK,), jnp.int32),
            pltpu.VMEM((CHUNK,), jnp.int32),
            pltpu.VMEM((CHUNK,), jnp.int32),
            pltpu.VMEM((CHUNK, d), jnp.float32),
            pltpu.VMEM((CHUNK, d), jnp.float32),
            pltpu.VMEM((CHUNK, d), jnp.float32),
            pltpu.VMEM((n_dst,), jnp.float32),
            pltpu.VMEM((NS * stripe,), jnp.float32),
            pltpu.VMEM((stripe, 8), jnp.float32),
            pltpu.SemaphoreType.DMA,
            pltpu.SemaphoreType.DMA,
            pltpu.SemaphoreType.DMA,
            pltpu.SemaphoreType.DMA,
            pltpu.SemaphoreType.DMA,
            pltpu.SemaphoreType.DMA,
            pltpu.SemaphoreType.DMA,
            pltpu.SemaphoreType.DMA,
            pltpu.SemaphoreType.DMA,
            pltpu.VMEM_SHARED((n_dst, d), jnp.float32),
        ) + tuple(pltpu.VMEM_SHARED((n_dst,), jnp.float32)
                  for _ in range(NS)),
        compiler_params=pltpu.CompilerParams(needs_layout_passes=False),
    )
    def body(x_h, il_h, batch_h, attr_h,
             summed_o, cnt_o, attr_o,
             src_a, src_b, src_c, dst_a, dst_b, dst_c,
             rows_a, rows_b, rows_c,
             hist_v, slab_v, col_v, g_a, g_b, g_c, s_a, s_b, s_c,
             i_a, i_b, i_c,
             summed_sh, *slots_sh):
        bidx_v, rows_v, sem = dst_a, rows_a, g_a
        srcs = (src_a, src_b, src_c)
        dsts = (dst_a, dst_b, dst_c)
        rows2 = (rows_a, rows_b, rows_c)
        gsems = (g_a, g_b, g_c)
        ssems = (s_a, s_b, s_c)
        isems = (i_a, i_b, i_c)
        c = lax.axis_index("c")
        s = lax.axis_index("s")
        wid = s * NC + c

        r0 = pl.multiple_of(s * stripe, stripe)
        z16 = jnp.zeros((16,), jnp.float32)

        def zrow_body(i, carry):
            rows_a[i, pl.ds(0, 16)] = z16
            for jq in range(1, d // 16):
                rows_a[i, pl.ds(pl.multiple_of(jq * 16, 16), 16)] = z16
            return carry

        lax.fori_loop(0, CHUNK, zrow_body, 0)

        @pl.when(s < NS - 1)
        def _zero_full():
            for q in range(stripe // CHUNK):
                pltpu.sync_copy(rows_a,
                                summed_sh.at[pl.ds(r0 + q * CHUNK, CHUNK)])

        @pl.when(s == NS - 1)
        def _zero_tail():
            t0 = (NS - 1) * stripe
            pltpu.sync_copy(rows_a, summed_sh.at[pl.ds(t0, CHUNK)])
            pltpu.sync_copy(rows_a.at[pl.ds(0, s_tail - CHUNK)],
                            summed_sh.at[pl.ds(t0 + CHUNK, s_tail - CHUNK)])

        def zero_body(i, carry):
            hist_v[pl.ds(pl.multiple_of(i * 16, 16), 16)] = z16
            return carry

        lax.fori_loop(0, n_dst // 16, zero_body, 0)
        plsc.subcore_barrier()

        c0 = (n_chunks * wid) // NW
        c1 = (n_chunks * (wid + 1)) // NW
        one16 = jnp.ones((16,), jnp.float32)

        def il_slice(i):
            base = pl.multiple_of(i * 2 * CHUNK, 2 * CHUNK)
            return il_h.at[pl.ds(base, 2 * CHUNK)]

        def fire_idx(i, b):
            pltpu.async_copy(il_slice(i), srcs[b], isems[b])

        def wait_idx(i, b):
            pltpu.make_async_copy(il_slice(i), srcs[b], isems[b]).wait()

        def fire_gather(b):
            pltpu.async_copy(x_h.at[srcs[b].at[pl.ds(0, CHUNK)]],
                             rows2[b], gsems[b])

        def wait_scatter(b):
            pltpu.make_async_copy(rows2[b], summed_sh.at[dsts[b]],
                                  ssems[b]).wait()

        pltpu.sync_copy(il_slice(c0), src_a)
        pltpu.sync_copy(il_slice(c0 + 1), src_b)
        fire_gather(0)
        fire_gather(1)
        fire_idx(c0 + 2, 2)

        def chunk_body(i, carry):
            for b in (0, 1, 2):
                @pl.when((i - c0) % 3 == b)
                def _():
                    pltpu.make_async_copy(x_h.at[srcs[b].at[pl.ds(0, CHUNK)]],
                                          rows2[b], gsems[b]).wait()
                    bn = (b + 2) % 3

                    @pl.when(i > c0)
                    def _drain():
                        wait_scatter(bn)

                    @pl.when(i + 2 < c1)
                    def _pref():
                        wait_idx(i + 2, bn)
                        fire_gather(bn)

                    for j in range(CHUNK // 16):
                        dv = srcs[b][pl.ds(CHUNK + j * 16, 16)]
                        plsc.addupdate_scatter(hist_v, [dv], one16)
                        dsts[b][pl.ds(j * 16, 16)] = dv
                    pltpu.async_copy(rows2[b], summed_sh.at[dsts[b]],
                                     ssems[b], add=True)

                    @pl.when(i + 3 < c1)
                    def _pref_idx():
                        fire_idx(i + 3, b)
            return carry

        lax.fori_loop(c0, c1, chunk_body, 0)
        for b in (0, 1, 2):
            @pl.when((c1 - 1 - c0) % 3 == b)
            def _final_drain():
                wait_scatter(b)
        for k in range(NS):
            @pl.when(s == k)
            def _pub_hist():
                pltpu.sync_copy(hist_v, slots_sh[k])
        plsc.subcore_barrier()

        def pub(r_lo, sz):
            pltpu.sync_copy(summed_sh.at[pl.ds(r_lo, sz)],
                            summed_o.at[c, pl.ds(r_lo, sz)])
            cps = [pltpu.async_copy(slots_sh[k].at[pl.ds(r_lo, sz)],
                                    slab_v.at[pl.ds(k * stripe, sz)], i_a)
                   for k in range(NS)]
            for cp in cps:
                cp.wait()
            for v in range(sz // 16):
                acc = slab_v[pl.ds(v * 16, 16)]
                for k in range(1, NS):
                    acc = acc + slab_v[pl.ds(k * stripe + v * 16, 16)]
                ridx = v * 16 + lax.iota(jnp.int32, 16)
                plsc.store_scatter(col_v, [ridx, jnp.zeros((16,), jnp.int32)],
                                   acc)
            pltpu.sync_copy(col_v.at[pl.ds(0, sz)],
                            cnt_o.at[c, pl.ds(r_lo, sz)])

        @pl.when(s < NS - 1)
        def _pub_full():
            pub(r0, stripe)

        @pl.when(s == NS - 1)
        def _pub_tail():
            pub((NS - 1) * stripe, s_tail)

        @pl.when(wid < NW - 1)
        def _full():
            b = pl.multiple_of(wid * 128, 128)
            pltpu.sync_copy(batch_h.at[pl.ds(b, 128)], bidx_v)
            pltpu.async_copy(attr_h.at[bidx_v], rows_v, sem).wait()
            pltpu.sync_copy(rows_v, attr_o.at[pl.ds(b, 128)])

        @pl.when(wid == NW - 1)
        def _tail():
            b = (NW - 1) * 128
            pltpu.sync_copy(batch_h.at[pl.ds(b, a_tail)],
                            bidx_v.at[pl.ds(0, a_tail)])
            pltpu.async_copy(attr_h.at[bidx_v.at[pl.ds(0, a_tail)]],
                             rows_v.at[pl.ds(0, a_tail)], sem).wait()
            pltpu.sync_copy(rows_v.at[pl.ds(0, a_tail)],
                            attr_o.at[pl.ds(b, a_tail)])

    return body(x, il, batch, attr)


def _tc_combine(summed2, cnt2, x_full, n_dst, attr_g, W_l, W_r, W_lin,
                b_l, b_lin):
    d = x_full.shape[1]
    blk = n_dst
    grid = n_dst // blk
    dn = (((1,), (1,)), ((), ()))

    def body(s2, c2, xt, ag, wl, wr, wlin, bl, blin, o):
        ssum = s2[0] + s2[1]
        cnt = c2[0] + c2[1]
        mean = ssum / jnp.maximum(cnt[:, 0:1], 1.0)
        acc = lax.dot_general(mean, wl[...], dn,
                              preferred_element_type=jnp.float32)
        acc = acc + lax.dot_general(xt[...], wr[...], dn,
                                    preferred_element_type=jnp.float32)
        acc = acc + 0.25 * lax.dot_general(ag[...], wlin[...], dn,
                                           preferred_element_type=jnp.float32)
        acc = acc + (bl[...] + 0.25 * blin[...])
        o[...] = jnp.maximum(acc, 0.0)

    return pl.pallas_call(
        body,
        grid=(grid,),
        in_specs=[
            pl.BlockSpec((NC, blk, d), lambda i: (0, i, 0)),
            pl.BlockSpec((NC, blk, 8), lambda i: (0, i, 0)),
            pl.BlockSpec((blk, d), lambda i: (i, 0)),
            pl.BlockSpec((blk, d), lambda i: (i, 0)),
            pl.BlockSpec((d, d), lambda i: (0, 0)),
            pl.BlockSpec((d, d), lambda i: (0, 0)),
            pl.BlockSpec((d, d), lambda i: (0, 0)),
            pl.BlockSpec((d,), lambda i: (0,)),
            pl.BlockSpec((d,), lambda i: (0,)),
        ],
        out_specs=pl.BlockSpec((blk, d), lambda i: (i, 0)),
        out_shape=jax.ShapeDtypeStruct((n_dst, d), jnp.float32),
    )(summed2, cnt2, x_full, attr_g, W_l, W_r, W_lin, b_l, b_lin)


def kernel(x, edge_index, batch, attr, W_l, b_l, W_r, W_lin, b_lin,
           size_src, size_dst):
    src = edge_index[0]
    dst = edge_index[1]
    n_dst = batch.shape[0]
    il = jnp.stack([src.reshape(-1, CHUNK), dst.reshape(-1, CHUNK)],
                   axis=1).reshape(-1)
    summed2, cnt2, attr_g = _sc_agg(x, il, batch, attr, n_dst)
    return _tc_combine(summed2, cnt2, x, n_dst, attr_g, W_l, W_r, W_lin,
                       b_l, b_lin)
